# baseline (speedup 1.0000x reference)
import functools

import jax
import jax.numpy as jnp
from jax import lax
from jax.experimental import pallas as pl
from jax.experimental.pallas import tpu as pltpu
from jax.experimental.pallas import tpu_sc as plsc

N = 10000
E = 160000
T = 64000
H = 128
NR = 6
NB = 8
NBLK = 2
SBF_DIM = 42

NC = 2
NSUB = 16
NW = NC * NSUB

BE = 640
BT = 512

_silu = jax.nn.silu


def _dot(a, b):
  return jnp.dot(a, b, preferred_element_type=jnp.float32)


def _node_embed_body(x_ref, f1w, f1b, bng, bnb, f2w, f2b, w1, w2,
                     a1_ref, a2_ref):
  x = x_ref[...]
  h1 = _dot(x, f1w[...]) + f1b[...]
  mu = jnp.mean(h1, axis=0, keepdims=True)
  var = jnp.mean((h1 - mu) ** 2, axis=0, keepdims=True)
  h1 = (h1 - mu) / jnp.sqrt(var + 1e-5) * bng[...] + bnb[...]
  h1 = jnp.maximum(h1, 0.0)
  h = jnp.maximum(_dot(h1, f2w[...]) + f2b[...], 0.0)
  a1_ref[...] = _dot(h, w1[...])
  a2_ref[...] = _dot(h, w2[...])


def _node_embed(x, p):
  return pl.pallas_call(
      _node_embed_body,
      out_shape=(jax.ShapeDtypeStruct((N, H), jnp.float32),
                 jax.ShapeDtypeStruct((N, H), jnp.float32)),
  )(x, p["f1"]["w"], p["f1"]["b"].reshape(1, -1),
    p["bng"].reshape(1, -1), p["bnb"].reshape(1, -1),
    p["f2"]["w"], p["f2"]["b"].reshape(1, -1),
    p["lin"]["w"][:H], p["lin"]["w"][H:2 * H])


def _edge_embed_body(s1_ref, s2_ref, rbf_ref, wr, br, w3, bl, wro,
                     m_ref, t0_ref):
  rbf = rbf_ref[...]
  r = _silu(_dot(rbf, wr[...]) + br[...])
  m = _silu(s1_ref[...] + s2_ref[...] + _dot(r, w3[...]) + bl[...])
  m_ref[...] = m
  t0_ref[...] = _dot(rbf, wro[...]) * m


def _edge_embed(s1, s2, rbf, p, wro):
  eb = lambda i: (i, 0)
  full = lambda i: (0, 0)
  return pl.pallas_call(
      _edge_embed_body,
      grid=(E // BE,),
      in_specs=[
          pl.BlockSpec((BE, H), eb),
          pl.BlockSpec((BE, H), eb),
          pl.BlockSpec((BE, NR), eb),
          pl.BlockSpec((NR, H), full),
          pl.BlockSpec((1, H), full),
          pl.BlockSpec((H, H), full),
          pl.BlockSpec((1, H), full),
          pl.BlockSpec((NR, H), full),
      ],
      out_specs=(pl.BlockSpec((BE, H), eb), pl.BlockSpec((BE, H), eb)),
      out_shape=(jax.ShapeDtypeStruct((E, H), jnp.float32),
                 jax.ShapeDtypeStruct((E, H), jnp.float32)),
  )(s1, s2, rbf, p["rbf"]["w"], p["rbf"]["b"].reshape(1, -1),
    p["lin"]["w"][2 * H:], p["lin"]["b"].reshape(1, -1), wro)


def _out_node_body(np_ref, l1w, l1b, l2w, l2b, l3w, l3b, wout, out_ref):
  node = np_ref[...]
  node = _silu(_dot(node, l1w[...]) + l1b[...])
  node = _silu(_dot(node, l2w[...]) + l2b[...])
  node = _silu(_dot(node, l3w[...]) + l3b[...])
  tot = jnp.sum(node, axis=0, keepdims=True)
  out_ref[...] = _dot(tot, wout[...])


def _out_node(node, p):
  ls = p["lins"]
  return pl.pallas_call(
      _out_node_body,
      out_shape=jax.ShapeDtypeStruct((1, 1), jnp.float32),
  )(node,
    ls[0]["w"], ls[0]["b"].reshape(1, -1),
    ls[1]["w"], ls[1]["b"].reshape(1, -1),
    ls[2]["w"], ls[2]["b"].reshape(1, -1),
    p["out"])


def _inter1_body(m_ref, rbf_ref, wkj, bkj, wji, bji, wr, xji_ref, xkjr_ref):
  m = m_ref[...]
  xji_ref[...] = _silu(_dot(m, wji[...]) + bji[...])
  rbfp = _dot(rbf_ref[...], wr[...])
  xkjr_ref[...] = _silu(_dot(m, wkj[...]) + bkj[...]) * rbfp


def _inter1(m, rbf, p):
  eb = lambda i: (i, 0)
  full = lambda i: (0, 0)
  return pl.pallas_call(
      _inter1_body,
      grid=(E // BE,),
      in_specs=[
          pl.BlockSpec((BE, H), eb),
          pl.BlockSpec((BE, NR), eb),
          pl.BlockSpec((H, H), full),
          pl.BlockSpec((1, H), full),
          pl.BlockSpec((H, H), full),
          pl.BlockSpec((1, H), full),
          pl.BlockSpec((NR, H), full),
      ],
      out_specs=(pl.BlockSpec((BE, H), eb), pl.BlockSpec((BE, H), eb)),
      out_shape=(jax.ShapeDtypeStruct((E, H), jnp.float32),
                 jax.ShapeDtypeStruct((E, H), jnp.float32)),
  )(m, rbf, p["kj"]["w"], p["kj"]["b"].reshape(1, -1),
    p["ji"]["w"], p["ji"]["b"].reshape(1, -1), p["rbf"]["w"])


def _inter2_body(xg_ref, sbf_ref, ws, wt, t_ref):
  sp = _dot(sbf_ref[...], ws[...])
  xg = xg_ref[...]
  acc = sp[:, 0:1] * _dot(xg, wt[0])
  for b in range(1, NB):
    acc = acc + sp[:, b:b + 1] * _dot(xg, wt[b])
  t_ref[...] = acc


def _inter2(xg, sbf, p):
  wt = jnp.transpose(p["W"], (1, 2, 0))
  tb = lambda i: (i, 0)
  return pl.pallas_call(
      _inter2_body,
      grid=(T // BT,),
      in_specs=[
          pl.BlockSpec((BT, H), tb),
          pl.BlockSpec((BT, SBF_DIM), tb),
          pl.BlockSpec((SBF_DIM, NB), lambda i: (0, 0)),
          pl.BlockSpec((NB, H, H), lambda i: (0, 0, 0)),
      ],
      out_specs=pl.BlockSpec((BT, H), tb),
      out_shape=jax.ShapeDtypeStruct((T, H), jnp.float32),
  )(xg, sbf, p["sbf"]["w"], wt)


def _inter3_body(xji_ref, agg_ref, m_ref, rbf_ref,
                 b1w, b1b, b2w, b2b, lw, lb,
                 a1w, a1b, a2w, a2b, a3w, a3b, a4w, a4b, wro,
                 mo_ref, tn_ref):
  h = xji_ref[...] + agg_ref[...]
  h = h + _silu(_dot(_silu(_dot(h, b1w[...]) + b1b[...]), b2w[...]) + b2b[...])
  h = _silu(_dot(h, lw[...]) + lb[...]) + m_ref[...]
  h = h + _silu(_dot(_silu(_dot(h, a1w[...]) + a1b[...]), a2w[...]) + a2b[...])
  h = h + _silu(_dot(_silu(_dot(h, a3w[...]) + a3b[...]), a4w[...]) + a4b[...])
  mo_ref[...] = h
  tn_ref[...] = _dot(rbf_ref[...], wro[...]) * h


def _inter3(xji, agg, m, rbf, p, wro):
  eb = lambda i: (i, 0)
  full = lambda i: (0, 0)
  wspec = pl.BlockSpec((H, H), full)
  bspec = pl.BlockSpec((1, H), full)
  bef = p["before"][0]
  af0, af1 = p["after"][0], p["after"][1]
  return pl.pallas_call(
      _inter3_body,
      grid=(E // BE,),
      in_specs=[
          pl.BlockSpec((BE, H), eb),
          pl.BlockSpec((BE, H), eb),
          pl.BlockSpec((BE, H), eb),
          pl.BlockSpec((BE, NR), eb),
          wspec, bspec, wspec, bspec, wspec, bspec,
          wspec, bspec, wspec, bspec, wspec, bspec, wspec, bspec,
          pl.BlockSpec((NR, H), full),
      ],
      out_specs=(pl.BlockSpec((BE, H), eb), pl.BlockSpec((BE, H), eb)),
      out_shape=(jax.ShapeDtypeStruct((E, H), jnp.float32),
                 jax.ShapeDtypeStruct((E, H), jnp.float32)),
  )(xji, agg, m, rbf,
    bef["l1"]["w"], bef["l1"]["b"].reshape(1, -1),
    bef["l2"]["w"], bef["l2"]["b"].reshape(1, -1),
    p["lin"]["w"], p["lin"]["b"].reshape(1, -1),
    af0["l1"]["w"], af0["l1"]["b"].reshape(1, -1),
    af0["l2"]["w"], af0["l2"]["b"].reshape(1, -1),
    af1["l1"]["w"], af1["l1"]["b"].reshape(1, -1),
    af1["l2"]["w"], af1["l2"]["b"].reshape(1, -1),
    wro)


@functools.lru_cache(maxsize=None)
def _sc_mesh():
  return plsc.VectorSubcoreMesh(core_axis_name="c", subcore_axis_name="s")


@functools.lru_cache(maxsize=None)
def _make_gather(m_rows, chunk):
  per_tile = m_rows // NW
  n_chunks = per_tile // chunk

  @functools.partial(
      pl.kernel,
      out_type=jax.ShapeDtypeStruct((m_rows, H), jnp.float32),
      mesh=_sc_mesh(),
      scratch_types=[
          pltpu.VMEM((chunk,), jnp.int32),
          pltpu.VMEM((chunk, H), jnp.float32),
          pltpu.SemaphoreType.DMA,
      ],
  )
  def k(table_hbm, idx_hbm, out_hbm, idx_v, rows_v, sem):
    wid = lax.axis_index("s") * NC + lax.axis_index("c")
    base = wid * per_tile

    def body(c, carry):
      off = base + c * chunk
      pltpu.sync_copy(idx_hbm.at[pl.ds(off, chunk)], idx_v)
      pltpu.async_copy(table_hbm.at[idx_v], rows_v, sem).wait()
      pltpu.sync_copy(rows_v, out_hbm.at[pl.ds(off, chunk)])
      return carry

    lax.fori_loop(0, n_chunks, body, 0)

  return k


@functools.lru_cache(maxsize=None)
def _make_scatter(m_rows, win, wpc, zb, chunk):
  per_tile = m_rows // NSUB
  n_chunks = per_tile // chunk
  wt_count = NSUB if (win // NSUB) % 8 == 0 else NSUB // 2
  rpt = win // wt_count
  n_zero = rpt // zb
  assert rpt % zb == 0 and per_tile % chunk == 0 and chunk % 16 == 0

  @functools.partial(
      pl.kernel,
      out_type=jax.ShapeDtypeStruct((2 * wpc * win, H), jnp.float32),
      mesh=_sc_mesh(),
      scratch_types=[
          pltpu.VMEM_SHARED((win + 8, H), jnp.float32),
          pltpu.VMEM((per_tile,), jnp.int32),
          pltpu.VMEM((chunk,), jnp.int32),
          pltpu.VMEM((chunk, H), jnp.float32),
      ],
  )
  def k(src_hbm, idx_hbm, zeros_hbm, out_hbm, shared, idx_all, dest_v, buf):
    cid = lax.axis_index("c")
    sid = lax.axis_index("s")
    src_base = sid * per_tile
    pltpu.sync_copy(idx_hbm.at[pl.ds(src_base, per_tile)], idx_all)

    def window(w, carry):
      lo = (cid * wpc + w) * win
      out_base = (cid * wpc + w) * win

      @pl.when(sid < wt_count)
      def _():
        def zero(z, zc):
          pltpu.sync_copy(zeros_hbm,
                          shared.at[pl.ds(sid * rpt + z * zb, zb)])
          return zc

        lax.fori_loop(0, n_zero, zero, 0)

      plsc.subcore_barrier()

      def scan(c, sc):
        off = c * chunk
        for q in range(chunk // 16):
          v = idx_all[pl.ds(off + q * 16, 16)] - lo
          ok = (v >= 0) & (v < win)
          dest_v[pl.ds(q * 16, 16)] = jnp.where(ok, v, win)
        pltpu.sync_copy(src_hbm.at[pl.ds(src_base + off, chunk)], buf)
        pltpu.sync_copy(buf, shared.at[dest_v], add=True)
        return sc

      lax.fori_loop(0, n_chunks, scan, 0)
      plsc.subcore_barrier()

      @pl.when(sid < wt_count)
      def _():
        pltpu.sync_copy(shared.at[pl.ds(sid * rpt, rpt)],
                        out_hbm.at[pl.ds(out_base + sid * rpt, rpt)])

      plsc.subcore_barrier()
      return carry

    lax.fori_loop(0, wpc, window, 0)

  return k


def _zeros_buf(zb):
  return jnp.zeros((zb, H), jnp.float32)


def _sc_gather(table, idx, m_rows, chunk):
  return _make_gather(m_rows, chunk)(table, idx)


def _segsum_edges(t, idx_ji):
  return _make_scatter(T, 8000, 10, 125, 80)(t, idx_ji, _zeros_buf(125))


def _segsum_nodes(t, i):
  out = _make_scatter(E, 5120, 1, 160, 80)(t, i, _zeros_buf(160))
  return out[:N]


def kernel(x, rbf, sbf, i, j, idx_kj, idx_ji, params):
  i = i.astype(jnp.int32)
  j = j.astype(jnp.int32)
  idx_kj = idx_kj.astype(jnp.int32)
  idx_ji = idx_ji.astype(jnp.int32)

  a1, a2 = _node_embed(x, params["emb"])
  s1 = _sc_gather(a1, i, E, 40)
  s2 = _sc_gather(a2, j, E, 40)
  m, t_out = _edge_embed(s1, s2, rbf, params["emb"],
                         params["out"][0]["rbf"]["w"])
  total = _out_node(_segsum_nodes(t_out, i), params["out"][0])
  for b in range(NBLK):
    pb = params["inter"][b]
    xji, xkjr = _inter1(m, rbf, pb)
    xg = _sc_gather(xkjr, idx_kj, T, 80)
    t = _inter2(xg, sbf, pb)
    agg = _segsum_edges(t, idx_ji)
    m, t_out = _inter3(xji, agg, m, rbf, pb,
                       params["out"][b + 1]["rbf"]["w"])
    total = total + _out_node(_segsum_nodes(t_out, i), params["out"][b + 1])
  return total.reshape((1,))


# speedup vs baseline: 1.4487x; 1.4487x over previous
import functools

import jax
import jax.numpy as jnp
from jax import lax
from jax.experimental import pallas as pl
from jax.experimental.pallas import tpu as pltpu
from jax.experimental.pallas import tpu_sc as plsc

N = 10000
E = 160000
T = 64000
H = 128
NR = 6
NB = 8
NBLK = 2
SBF_DIM = 42

NC = 2
NSUB = 16
NW = NC * NSUB

BE = 640
BT = 512

_silu = jax.nn.silu


def _dot(a, b):
  return jnp.dot(a, b, preferred_element_type=jnp.float32)


def _node_embed_body(x_ref, f1w, f1b, bng, bnb, f2w, f2b, w1, w2,
                     a1_ref, a2_ref):
  x = x_ref[...]
  h1 = _dot(x, f1w[...]) + f1b[...]
  mu = jnp.mean(h1, axis=0, keepdims=True)
  var = jnp.mean((h1 - mu) ** 2, axis=0, keepdims=True)
  h1 = (h1 - mu) / jnp.sqrt(var + 1e-5) * bng[...] + bnb[...]
  h1 = jnp.maximum(h1, 0.0)
  h = jnp.maximum(_dot(h1, f2w[...]) + f2b[...], 0.0)
  a1_ref[...] = _dot(h, w1[...])
  a2_ref[...] = _dot(h, w2[...])


def _node_embed(x, p):
  return pl.pallas_call(
      _node_embed_body,
      out_shape=(jax.ShapeDtypeStruct((N, H), jnp.float32),
                 jax.ShapeDtypeStruct((N, H), jnp.float32)),
  )(x, p["f1"]["w"], p["f1"]["b"].reshape(1, -1),
    p["bng"].reshape(1, -1), p["bnb"].reshape(1, -1),
    p["f2"]["w"], p["f2"]["b"].reshape(1, -1),
    p["lin"]["w"][:H], p["lin"]["w"][H:2 * H])


def _edge_embed_body(s1_ref, s2_ref, rbf_ref, wr, br, w3, bl, wro,
                     m_ref, t0_ref):
  rbf = rbf_ref[...]
  r = _silu(_dot(rbf, wr[...]) + br[...])
  m = _silu(s1_ref[...] + s2_ref[...] + _dot(r, w3[...]) + bl[...])
  m_ref[...] = m
  t0_ref[...] = _dot(rbf, wro[...]) * m


def _edge_embed(s1, s2, rbf, p, wro):
  eb = lambda i: (i, 0)
  full = lambda i: (0, 0)
  return pl.pallas_call(
      _edge_embed_body,
      grid=(E // BE,),
      in_specs=[
          pl.BlockSpec((BE, H), eb),
          pl.BlockSpec((BE, H), eb),
          pl.BlockSpec((BE, NR), eb),
          pl.BlockSpec((NR, H), full),
          pl.BlockSpec((1, H), full),
          pl.BlockSpec((H, H), full),
          pl.BlockSpec((1, H), full),
          pl.BlockSpec((NR, H), full),
      ],
      out_specs=(pl.BlockSpec((BE, H), eb), pl.BlockSpec((BE, H), eb)),
      out_shape=(jax.ShapeDtypeStruct((E, H), jnp.float32),
                 jax.ShapeDtypeStruct((E, H), jnp.float32)),
  )(s1, s2, rbf, p["rbf"]["w"], p["rbf"]["b"].reshape(1, -1),
    p["lin"]["w"][2 * H:], p["lin"]["b"].reshape(1, -1), wro)


def _out_node_body(np_ref, l1w, l1b, l2w, l2b, l3w, l3b, wout, out_ref):
  node = np_ref[...]
  node = _silu(_dot(node, l1w[...]) + l1b[...])
  node = _silu(_dot(node, l2w[...]) + l2b[...])
  node = _silu(_dot(node, l3w[...]) + l3b[...])
  tot = jnp.sum(node, axis=0, keepdims=True)
  out_ref[...] = _dot(tot, wout[...])


def _out_node(node, p):
  ls = p["lins"]
  return pl.pallas_call(
      _out_node_body,
      out_shape=jax.ShapeDtypeStruct((1, 1), jnp.float32),
  )(node,
    ls[0]["w"], ls[0]["b"].reshape(1, -1),
    ls[1]["w"], ls[1]["b"].reshape(1, -1),
    ls[2]["w"], ls[2]["b"].reshape(1, -1),
    p["out"])


def _inter1_body(m_ref, rbf_ref, wkj, bkj, wji, bji, wr, xji_ref, xkjr_ref):
  m = m_ref[...]
  xji_ref[...] = _silu(_dot(m, wji[...]) + bji[...])
  rbfp = _dot(rbf_ref[...], wr[...])
  xkjr_ref[...] = _silu(_dot(m, wkj[...]) + bkj[...]) * rbfp


def _inter1(m, rbf, p):
  eb = lambda i: (i, 0)
  full = lambda i: (0, 0)
  return pl.pallas_call(
      _inter1_body,
      grid=(E // BE,),
      in_specs=[
          pl.BlockSpec((BE, H), eb),
          pl.BlockSpec((BE, NR), eb),
          pl.BlockSpec((H, H), full),
          pl.BlockSpec((1, H), full),
          pl.BlockSpec((H, H), full),
          pl.BlockSpec((1, H), full),
          pl.BlockSpec((NR, H), full),
      ],
      out_specs=(pl.BlockSpec((BE, H), eb), pl.BlockSpec((BE, H), eb)),
      out_shape=(jax.ShapeDtypeStruct((E, H), jnp.float32),
                 jax.ShapeDtypeStruct((E, H), jnp.float32)),
  )(m, rbf, p["kj"]["w"], p["kj"]["b"].reshape(1, -1),
    p["ji"]["w"], p["ji"]["b"].reshape(1, -1), p["rbf"]["w"])


def _inter2_body(xg_ref, sbf_ref, ws, wt, t_ref):
  sp = _dot(sbf_ref[...], ws[...])
  xg = xg_ref[...]
  acc = sp[:, 0:1] * _dot(xg, wt[0])
  for b in range(1, NB):
    acc = acc + sp[:, b:b + 1] * _dot(xg, wt[b])
  t_ref[...] = acc


def _inter2(xg, sbf, p):
  wt = jnp.transpose(p["W"], (1, 2, 0))
  tb = lambda i: (i, 0)
  return pl.pallas_call(
      _inter2_body,
      grid=(T // BT,),
      in_specs=[
          pl.BlockSpec((BT, H), tb),
          pl.BlockSpec((BT, SBF_DIM), tb),
          pl.BlockSpec((SBF_DIM, NB), lambda i: (0, 0)),
          pl.BlockSpec((NB, H, H), lambda i: (0, 0, 0)),
      ],
      out_specs=pl.BlockSpec((BT, H), tb),
      out_shape=jax.ShapeDtypeStruct((T, H), jnp.float32),
  )(xg, sbf, p["sbf"]["w"], wt)


def _inter3_body(xji_ref, agg_ref, m_ref, rbf_ref,
                 b1w, b1b, b2w, b2b, lw, lb,
                 a1w, a1b, a2w, a2b, a3w, a3b, a4w, a4b, wro,
                 mo_ref, tn_ref):
  h = xji_ref[...] + agg_ref[...]
  h = h + _silu(_dot(_silu(_dot(h, b1w[...]) + b1b[...]), b2w[...]) + b2b[...])
  h = _silu(_dot(h, lw[...]) + lb[...]) + m_ref[...]
  h = h + _silu(_dot(_silu(_dot(h, a1w[...]) + a1b[...]), a2w[...]) + a2b[...])
  h = h + _silu(_dot(_silu(_dot(h, a3w[...]) + a3b[...]), a4w[...]) + a4b[...])
  mo_ref[...] = h
  tn_ref[...] = _dot(rbf_ref[...], wro[...]) * h


def _inter3(xji, agg, m, rbf, p, wro):
  eb = lambda i: (i, 0)
  full = lambda i: (0, 0)
  wspec = pl.BlockSpec((H, H), full)
  bspec = pl.BlockSpec((1, H), full)
  bef = p["before"][0]
  af0, af1 = p["after"][0], p["after"][1]
  return pl.pallas_call(
      _inter3_body,
      grid=(E // BE,),
      in_specs=[
          pl.BlockSpec((BE, H), eb),
          pl.BlockSpec((BE, H), eb),
          pl.BlockSpec((BE, H), eb),
          pl.BlockSpec((BE, NR), eb),
          wspec, bspec, wspec, bspec, wspec, bspec,
          wspec, bspec, wspec, bspec, wspec, bspec, wspec, bspec,
          pl.BlockSpec((NR, H), full),
      ],
      out_specs=(pl.BlockSpec((BE, H), eb), pl.BlockSpec((BE, H), eb)),
      out_shape=(jax.ShapeDtypeStruct((E, H), jnp.float32),
                 jax.ShapeDtypeStruct((E, H), jnp.float32)),
  )(xji, agg, m, rbf,
    bef["l1"]["w"], bef["l1"]["b"].reshape(1, -1),
    bef["l2"]["w"], bef["l2"]["b"].reshape(1, -1),
    p["lin"]["w"], p["lin"]["b"].reshape(1, -1),
    af0["l1"]["w"], af0["l1"]["b"].reshape(1, -1),
    af0["l2"]["w"], af0["l2"]["b"].reshape(1, -1),
    af1["l1"]["w"], af1["l1"]["b"].reshape(1, -1),
    af1["l2"]["w"], af1["l2"]["b"].reshape(1, -1),
    wro)


@functools.lru_cache(maxsize=None)
def _sc_mesh():
  return plsc.VectorSubcoreMesh(core_axis_name="c", subcore_axis_name="s")


@functools.lru_cache(maxsize=None)
def _make_gather(m_rows, chunk):
  per_tile = m_rows // NW
  n_chunks = per_tile // chunk

  @functools.partial(
      pl.kernel,
      out_type=jax.ShapeDtypeStruct((m_rows, H), jnp.float32),
      mesh=_sc_mesh(),
      scratch_types=[
          pltpu.VMEM((chunk,), jnp.int32),
          pltpu.VMEM((chunk, H), jnp.float32),
          pltpu.SemaphoreType.DMA,
      ],
  )
  def k(table_hbm, idx_hbm, out_hbm, idx_v, rows_v, sem):
    wid = lax.axis_index("s") * NC + lax.axis_index("c")
    base = wid * per_tile

    def body(c, carry):
      off = base + c * chunk
      pltpu.sync_copy(idx_hbm.at[pl.ds(off, chunk)], idx_v)
      pltpu.async_copy(table_hbm.at[idx_v], rows_v, sem).wait()
      pltpu.sync_copy(rows_v, out_hbm.at[pl.ds(off, chunk)])
      return carry

    lax.fori_loop(0, n_chunks, body, 0)

  return k


@functools.lru_cache(maxsize=None)
def _make_scatter(m_rows, win, wpc, zb, chunk):
  per_tile = m_rows // NSUB
  n_groups = per_tile // 16
  cap = per_tile + chunk + 16
  wt_count = NSUB if (win // NSUB) % 8 == 0 else NSUB // 2
  rpt = win // wt_count
  n_zero = rpt // zb
  assert rpt % zb == 0 and per_tile % 16 == 0 and chunk % 16 == 0

  @functools.partial(
      pl.kernel,
      out_type=jax.ShapeDtypeStruct((2 * wpc * win, H), jnp.float32),
      mesh=_sc_mesh(),
      compiler_params=pltpu.CompilerParams(needs_layout_passes=False),
      scratch_types=[
          pltpu.VMEM_SHARED((win + 8, H), jnp.float32),
          pltpu.VMEM((per_tile,), jnp.int32),
          pltpu.VMEM((cap,), jnp.int32),
          pltpu.VMEM((cap,), jnp.int32),
          pltpu.VMEM((chunk,), jnp.int32),
          pltpu.VMEM((chunk,), jnp.int32),
          pltpu.VMEM((chunk, H), jnp.float32),
          pltpu.SemaphoreType.DMA,
      ],
  )
  def k(src_hbm, idx_hbm, zeros_hbm, out_hbm, shared, idx_all,
        dest_list, tid_list, dest_v, tid_v, buf, sem):
    cid = lax.axis_index("c")
    sid = lax.axis_index("s")
    src_base = sid * per_tile
    pltpu.sync_copy(idx_hbm.at[pl.ds(src_base, per_tile)], idx_all)
    iota16 = lax.iota(jnp.int32, 16)

    def window(w, carry):
      lo = (cid * wpc + w) * win
      out_base = (cid * wpc + w) * win

      @pl.when(sid < wt_count)
      def _():
        def zero(z, zc):
          pltpu.sync_copy(zeros_hbm,
                          shared.at[pl.ds(sid * rpt + z * zb, zb)])
          return zc

        lax.fori_loop(0, n_zero, zero, 0)

      plsc.subcore_barrier()

      def grp(g, cnt):
        v = idx_all[pl.ds(g * 16, 16)] - lo
        ok = ((v >= 0) & (v < win)).astype(jnp.int32)
        pre = plsc.cumsum(ok)
        pos = jnp.where(ok > 0, cnt + pre - ok, (cap - 16) + iota16)
        plsc.store_scatter(dest_list, [pos], v)
        plsc.store_scatter(tid_list, [pos], iota16 + (src_base + g * 16))
        return cnt + jnp.sum(ok)

      cnt = lax.fori_loop(0, n_groups, grp, 0)
      for kp in range(chunk // 16):
        pos = cnt + kp * 16 + iota16
        plsc.store_scatter(dest_list, [pos], jnp.full((16,), win, jnp.int32))
        plsc.store_scatter(tid_list, [pos],
                           jnp.full((16,), src_base, jnp.int32))

      def chunkf(c, cc):
        @pl.when(c * chunk < cnt)
        def _():
          off = c * chunk
          for q in range(chunk // 16):
            dest_v[pl.ds(q * 16, 16)] = dest_list[pl.ds(off + q * 16, 16)]
            tid_v[pl.ds(q * 16, 16)] = tid_list[pl.ds(off + q * 16, 16)]
          pltpu.async_copy(src_hbm.at[tid_v], buf, sem).wait()
          pltpu.sync_copy(buf, shared.at[dest_v], add=True)

        return cc

      lax.fori_loop(0, (per_tile + chunk - 1) // chunk, chunkf, 0)
      plsc.subcore_barrier()

      @pl.when(sid < wt_count)
      def _():
        pltpu.sync_copy(shared.at[pl.ds(sid * rpt, rpt)],
                        out_hbm.at[pl.ds(out_base + sid * rpt, rpt)])

      plsc.subcore_barrier()
      return carry

    lax.fori_loop(0, wpc, window, 0)

  return k


def _zeros_buf(zb):
  return jnp.zeros((zb, H), jnp.float32)


def _sc_gather(table, idx, m_rows, chunk):
  return _make_gather(m_rows, chunk)(table, idx)


def _segsum_edges(t, idx_ji):
  return _make_scatter(T, 8000, 10, 125, 80)(t, idx_ji, _zeros_buf(125))


def _segsum_nodes(t, i):
  out = _make_scatter(E, 5120, 1, 160, 80)(t, i, _zeros_buf(160))
  return out[:N]


def kernel(x, rbf, sbf, i, j, idx_kj, idx_ji, params):
  i = i.astype(jnp.int32)
  j = j.astype(jnp.int32)
  idx_kj = idx_kj.astype(jnp.int32)
  idx_ji = idx_ji.astype(jnp.int32)

  a1, a2 = _node_embed(x, params["emb"])
  s1 = _sc_gather(a1, i, E, 40)
  s2 = _sc_gather(a2, j, E, 40)
  m, t_out = _edge_embed(s1, s2, rbf, params["emb"],
                         params["out"][0]["rbf"]["w"])
  total = _out_node(_segsum_nodes(t_out, i), params["out"][0])
  for b in range(NBLK):
    pb = params["inter"][b]
    xji, xkjr = _inter1(m, rbf, pb)
    xg = _sc_gather(xkjr, idx_kj, T, 80)
    t = _inter2(xg, sbf, pb)
    agg = _segsum_edges(t, idx_ji)
    m, t_out = _inter3(xji, agg, m, rbf, pb,
                       params["out"][b + 1]["rbf"]["w"])
    total = total + _out_node(_segsum_nodes(t_out, i), params["out"][b + 1])
  return total.reshape((1,))


# speedup vs baseline: 1.6870x; 1.1645x over previous
import functools

import jax
import jax.numpy as jnp
from jax import lax
from jax.experimental import pallas as pl
from jax.experimental.pallas import tpu as pltpu
from jax.experimental.pallas import tpu_sc as plsc

N = 10000
E = 160000
T = 64000
H = 128
NR = 6
NB = 8
NBLK = 2
SBF_DIM = 42

NC = 2
NSUB = 16
NW = NC * NSUB

BE = 640
BT = 512

_silu = jax.nn.silu


def _dot(a, b):
  return jnp.dot(a, b, preferred_element_type=jnp.float32)


def _node_embed_body(x_ref, f1w, f1b, bng, bnb, f2w, f2b, w1, w2,
                     a1_ref, a2_ref):
  x = x_ref[...]
  h1 = _dot(x, f1w[...]) + f1b[...]
  mu = jnp.mean(h1, axis=0, keepdims=True)
  var = jnp.mean((h1 - mu) ** 2, axis=0, keepdims=True)
  h1 = (h1 - mu) / jnp.sqrt(var + 1e-5) * bng[...] + bnb[...]
  h1 = jnp.maximum(h1, 0.0)
  h = jnp.maximum(_dot(h1, f2w[...]) + f2b[...], 0.0)
  a1_ref[...] = _dot(h, w1[...])
  a2_ref[...] = _dot(h, w2[...])


def _node_embed(x, p):
  return pl.pallas_call(
      _node_embed_body,
      out_shape=(jax.ShapeDtypeStruct((N, H), jnp.float32),
                 jax.ShapeDtypeStruct((N, H), jnp.float32)),
  )(x, p["f1"]["w"], p["f1"]["b"].reshape(1, -1),
    p["bng"].reshape(1, -1), p["bnb"].reshape(1, -1),
    p["f2"]["w"], p["f2"]["b"].reshape(1, -1),
    p["lin"]["w"][:H], p["lin"]["w"][H:2 * H])


def _edge_embed_body(s1_ref, s2_ref, rbf_ref, wr, br, w3, bl, wro,
                     m_ref, t0_ref):
  rbf = rbf_ref[...]
  r = _silu(_dot(rbf, wr[...]) + br[...])
  m = _silu(s1_ref[...] + s2_ref[...] + _dot(r, w3[...]) + bl[...])
  m_ref[...] = m
  t0_ref[...] = _dot(rbf, wro[...]) * m


def _edge_embed(s1, s2, rbf, p, wro):
  eb = lambda i: (i, 0)
  full = lambda i: (0, 0)
  return pl.pallas_call(
      _edge_embed_body,
      grid=(E // BE,),
      in_specs=[
          pl.BlockSpec((BE, H), eb),
          pl.BlockSpec((BE, H), eb),
          pl.BlockSpec((BE, NR), eb),
          pl.BlockSpec((NR, H), full),
          pl.BlockSpec((1, H), full),
          pl.BlockSpec((H, H), full),
          pl.BlockSpec((1, H), full),
          pl.BlockSpec((NR, H), full),
      ],
      out_specs=(pl.BlockSpec((BE, H), eb), pl.BlockSpec((BE, H), eb)),
      out_shape=(jax.ShapeDtypeStruct((E, H), jnp.float32),
                 jax.ShapeDtypeStruct((E, H), jnp.float32)),
  )(s1, s2, rbf, p["rbf"]["w"], p["rbf"]["b"].reshape(1, -1),
    p["lin"]["w"][2 * H:], p["lin"]["b"].reshape(1, -1), wro)


def _out_node_body(np_ref, l1w, l1b, l2w, l2b, l3w, l3b, wout, out_ref):
  node = np_ref[...]
  node = _silu(_dot(node, l1w[...]) + l1b[...])
  node = _silu(_dot(node, l2w[...]) + l2b[...])
  node = _silu(_dot(node, l3w[...]) + l3b[...])
  tot = jnp.sum(node, axis=0, keepdims=True)
  out_ref[...] = _dot(tot, wout[...])


def _out_node(node, p):
  ls = p["lins"]
  return pl.pallas_call(
      _out_node_body,
      out_shape=jax.ShapeDtypeStruct((1, 1), jnp.float32),
  )(node,
    ls[0]["w"], ls[0]["b"].reshape(1, -1),
    ls[1]["w"], ls[1]["b"].reshape(1, -1),
    ls[2]["w"], ls[2]["b"].reshape(1, -1),
    p["out"])


def _inter1_body(m_ref, rbf_ref, wkj, bkj, wji, bji, wr, xji_ref, xkjr_ref):
  m = m_ref[...]
  xji_ref[...] = _silu(_dot(m, wji[...]) + bji[...])
  rbfp = _dot(rbf_ref[...], wr[...])
  xkjr_ref[...] = _silu(_dot(m, wkj[...]) + bkj[...]) * rbfp


def _inter1(m, rbf, p):
  eb = lambda i: (i, 0)
  full = lambda i: (0, 0)
  return pl.pallas_call(
      _inter1_body,
      grid=(E // BE,),
      in_specs=[
          pl.BlockSpec((BE, H), eb),
          pl.BlockSpec((BE, NR), eb),
          pl.BlockSpec((H, H), full),
          pl.BlockSpec((1, H), full),
          pl.BlockSpec((H, H), full),
          pl.BlockSpec((1, H), full),
          pl.BlockSpec((NR, H), full),
      ],
      out_specs=(pl.BlockSpec((BE, H), eb), pl.BlockSpec((BE, H), eb)),
      out_shape=(jax.ShapeDtypeStruct((E, H), jnp.float32),
                 jax.ShapeDtypeStruct((E, H), jnp.float32)),
  )(m, rbf, p["kj"]["w"], p["kj"]["b"].reshape(1, -1),
    p["ji"]["w"], p["ji"]["b"].reshape(1, -1), p["rbf"]["w"])


def _inter2_body(xg_ref, sbf_ref, ws, wt, t_ref):
  sp = _dot(sbf_ref[...], ws[...])
  xg = xg_ref[...]
  acc = sp[:, 0:1] * _dot(xg, wt[0])
  for b in range(1, NB):
    acc = acc + sp[:, b:b + 1] * _dot(xg, wt[b])
  t_ref[...] = acc


def _inter2(xg, sbf, p):
  wt = jnp.transpose(p["W"], (1, 2, 0))
  tb = lambda i: (i, 0)
  return pl.pallas_call(
      _inter2_body,
      grid=(T // BT,),
      in_specs=[
          pl.BlockSpec((BT, H), tb),
          pl.BlockSpec((BT, SBF_DIM), tb),
          pl.BlockSpec((SBF_DIM, NB), lambda i: (0, 0)),
          pl.BlockSpec((NB, H, H), lambda i: (0, 0, 0)),
      ],
      out_specs=pl.BlockSpec((BT, H), tb),
      out_shape=jax.ShapeDtypeStruct((T, H), jnp.float32),
  )(xg, sbf, p["sbf"]["w"], wt)


def _inter3_body(xji_ref, agg_ref, m_ref, rbf_ref,
                 b1w, b1b, b2w, b2b, lw, lb,
                 a1w, a1b, a2w, a2b, a3w, a3b, a4w, a4b, wro,
                 mo_ref, tn_ref):
  h = xji_ref[...] + agg_ref[...]
  h = h + _silu(_dot(_silu(_dot(h, b1w[...]) + b1b[...]), b2w[...]) + b2b[...])
  h = _silu(_dot(h, lw[...]) + lb[...]) + m_ref[...]
  h = h + _silu(_dot(_silu(_dot(h, a1w[...]) + a1b[...]), a2w[...]) + a2b[...])
  h = h + _silu(_dot(_silu(_dot(h, a3w[...]) + a3b[...]), a4w[...]) + a4b[...])
  mo_ref[...] = h
  tn_ref[...] = _dot(rbf_ref[...], wro[...]) * h


def _inter3(xji, agg, m, rbf, p, wro):
  eb = lambda i: (i, 0)
  full = lambda i: (0, 0)
  wspec = pl.BlockSpec((H, H), full)
  bspec = pl.BlockSpec((1, H), full)
  bef = p["before"][0]
  af0, af1 = p["after"][0], p["after"][1]
  return pl.pallas_call(
      _inter3_body,
      grid=(E // BE,),
      in_specs=[
          pl.BlockSpec((BE, H), eb),
          pl.BlockSpec((BE, H), eb),
          pl.BlockSpec((BE, H), eb),
          pl.BlockSpec((BE, NR), eb),
          wspec, bspec, wspec, bspec, wspec, bspec,
          wspec, bspec, wspec, bspec, wspec, bspec, wspec, bspec,
          pl.BlockSpec((NR, H), full),
      ],
      out_specs=(pl.BlockSpec((BE, H), eb), pl.BlockSpec((BE, H), eb)),
      out_shape=(jax.ShapeDtypeStruct((E, H), jnp.float32),
                 jax.ShapeDtypeStruct((E, H), jnp.float32)),
  )(xji, agg, m, rbf,
    bef["l1"]["w"], bef["l1"]["b"].reshape(1, -1),
    bef["l2"]["w"], bef["l2"]["b"].reshape(1, -1),
    p["lin"]["w"], p["lin"]["b"].reshape(1, -1),
    af0["l1"]["w"], af0["l1"]["b"].reshape(1, -1),
    af0["l2"]["w"], af0["l2"]["b"].reshape(1, -1),
    af1["l1"]["w"], af1["l1"]["b"].reshape(1, -1),
    af1["l2"]["w"], af1["l2"]["b"].reshape(1, -1),
    wro)


@functools.lru_cache(maxsize=None)
def _sc_mesh():
  return plsc.VectorSubcoreMesh(core_axis_name="c", subcore_axis_name="s")


@functools.lru_cache(maxsize=None)
def _make_gather(m_rows, chunk, kq):
  per_tile = m_rows // NW
  n_chunks = per_tile // chunk
  assert n_chunks % kq == 0

  @functools.partial(
      pl.kernel,
      out_type=jax.ShapeDtypeStruct((m_rows, H), jnp.float32),
      mesh=_sc_mesh(),
      scratch_types=[
          pltpu.VMEM((per_tile,), jnp.int32),
          pltpu.VMEM((kq, chunk, H), jnp.float32),
          pltpu.SemaphoreType.DMA,
          pltpu.SemaphoreType.DMA,
      ],
  )
  def k(table_hbm, idx_hbm, out_hbm, idx_all, rows, sem_g, sem_w):
    wid = lax.axis_index("s") * NC + lax.axis_index("c")
    base = wid * per_tile
    pltpu.sync_copy(idx_hbm.at[pl.ds(base, per_tile)], idx_all)

    def group(g, carry):
      gd = []
      for b in range(kq):
        off = (g * kq + b) * chunk
        gd.append(pltpu.async_copy(
            table_hbm.at[idx_all.at[pl.ds(off, chunk)]], rows.at[b], sem_g))
      for d in gd:
        d.wait()
      wd = []
      for b in range(kq):
        off = (g * kq + b) * chunk
        wd.append(pltpu.async_copy(
            rows.at[b], out_hbm.at[pl.ds(base + off, chunk)], sem_w))
      for d in wd:
        d.wait()
      return carry

    lax.fori_loop(0, n_chunks // kq, group, 0)

  return k


@functools.lru_cache(maxsize=None)
def _make_scatter(m_rows, win, wpc, zb, chunk, kb):
  per_tile = m_rows // NSUB
  n_groups = per_tile // 16
  cap = per_tile + chunk + 16
  wt_count = NSUB if (win // NSUB) % 8 == 0 else NSUB // 2
  rpt = win // wt_count
  n_zero = rpt // zb
  assert rpt % zb == 0 and per_tile % 16 == 0 and chunk % 16 == 0

  @functools.partial(
      pl.kernel,
      out_type=jax.ShapeDtypeStruct((2 * wpc * win, H), jnp.float32),
      mesh=_sc_mesh(),
      compiler_params=pltpu.CompilerParams(needs_layout_passes=False),
      scratch_types=[
          pltpu.VMEM_SHARED((win + 8, H), jnp.float32),
          pltpu.VMEM((per_tile,), jnp.int32),
          pltpu.VMEM((cap,), jnp.int32),
          pltpu.VMEM((cap,), jnp.int32),
          pltpu.VMEM((chunk,), jnp.int32),
          pltpu.VMEM((kb, chunk, H), jnp.float32),
          pltpu.SemaphoreType.DMA,
          pltpu.SemaphoreType.DMA,
      ],
  )
  def k(src_hbm, idx_hbm, zeros_hbm, out_hbm, shared, idx_all,
        dest_list, tid_list, dest_v, bufs, sem_g, sem_z):
    cid = lax.axis_index("c")
    sid = lax.axis_index("s")
    src_base = sid * per_tile
    pltpu.sync_copy(idx_hbm.at[pl.ds(src_base, per_tile)], idx_all)
    iota16 = lax.iota(jnp.int32, 16)

    def window(w, carry):
      lo = (cid * wpc + w) * win
      out_base = (cid * wpc + w) * win

      @pl.when(sid < wt_count)
      def _():
        zd = [pltpu.async_copy(
            zeros_hbm, shared.at[pl.ds(sid * rpt + z * zb, zb)], sem_z)
            for z in range(n_zero)]
        for d in zd:
          d.wait()

      plsc.subcore_barrier()

      def grp(g, cnt):
        v = idx_all[pl.ds(g * 16, 16)] - lo
        ok = ((v >= 0) & (v < win)).astype(jnp.int32)
        pre = plsc.cumsum(ok)
        pos = jnp.where(ok > 0, cnt + pre - ok, (cap - 16) + iota16)
        plsc.store_scatter(dest_list, [pos], v)
        plsc.store_scatter(tid_list, [pos], iota16 + (src_base + g * 16))
        return cnt + jnp.sum(ok)

      cnt = lax.fori_loop(0, n_groups, grp, 0)
      for kp in range(chunk // 16):
        pos = cnt + kp * 16 + iota16
        plsc.store_scatter(dest_list, [pos], jnp.full((16,), win, jnp.int32))
        plsc.store_scatter(tid_list, [pos],
                           jnp.full((16,), src_base, jnp.int32))

      n_slots = (per_tile + chunk - 1) // chunk
      n_cgroups = (n_slots + kb - 1) // kb

      def cgroup(g2, cc):
        for b in range(kb):
          c = g2 * kb + b
          if True:
            @pl.when(c * chunk < cnt)
            def _(c=c, b=b):
              off = c * chunk
              pltpu.async_copy(
                  src_hbm.at[tid_list.at[pl.ds(off, chunk)]],
                  bufs.at[b], sem_g)

        for b in range(kb):
          c = g2 * kb + b
          if True:
            @pl.when(c * chunk < cnt)
            def _(c=c, b=b):
              pltpu.make_async_copy(
                  src_hbm.at[pl.ds(0, chunk)], bufs.at[b], sem_g).wait()

        for b in range(kb):
          c = g2 * kb + b
          if True:
            @pl.when(c * chunk < cnt)
            def _(c=c, b=b):
              off = c * chunk
              for q in range(chunk // 16):
                dest_v[pl.ds(q * 16, 16)] = dest_list[pl.ds(off + q * 16, 16)]
              pltpu.sync_copy(bufs.at[b], shared.at[dest_v], add=True)

        return cc

      lax.fori_loop(0, n_cgroups, cgroup, 0)
      plsc.subcore_barrier()

      @pl.when(sid < wt_count)
      def _():
        pltpu.sync_copy(shared.at[pl.ds(sid * rpt, rpt)],
                        out_hbm.at[pl.ds(out_base + sid * rpt, rpt)])

      plsc.subcore_barrier()
      return carry

    lax.fori_loop(0, wpc, window, 0)

  return k


def _zeros_buf(zb):
  return jnp.zeros((zb, H), jnp.float32)


def _sc_gather(table, idx, m_rows, chunk):
  return _make_gather(m_rows, chunk, 5)(table, idx)


def _segsum_edges(t, idx_ji):
  return _make_scatter(T, 8000, 10, 125, 80, 5)(t, idx_ji, _zeros_buf(125))


def _segsum_nodes(t, i):
  out = _make_scatter(E, 5120, 1, 160, 80, 5)(t, i, _zeros_buf(160))
  return out[:N]


def kernel(x, rbf, sbf, i, j, idx_kj, idx_ji, params):
  i = i.astype(jnp.int32)
  j = j.astype(jnp.int32)
  idx_kj = idx_kj.astype(jnp.int32)
  idx_ji = idx_ji.astype(jnp.int32)

  a1, a2 = _node_embed(x, params["emb"])
  s1 = _sc_gather(a1, i, E, 40)
  s2 = _sc_gather(a2, j, E, 40)
  m, t_out = _edge_embed(s1, s2, rbf, params["emb"],
                         params["out"][0]["rbf"]["w"])
  total = _out_node(_segsum_nodes(t_out, i), params["out"][0])
  for b in range(NBLK):
    pb = params["inter"][b]
    xji, xkjr = _inter1(m, rbf, pb)
    xg = _sc_gather(xkjr, idx_kj, T, 80)
    t = _inter2(xg, sbf, pb)
    agg = _segsum_edges(t, idx_ji)
    m, t_out = _inter3(xji, agg, m, rbf, pb,
                       params["out"][b + 1]["rbf"]["w"])
    total = total + _out_node(_segsum_nodes(t_out, i), params["out"][b + 1])
  return total.reshape((1,))


# speedup vs baseline: 1.7702x; 1.0493x over previous
import functools

import jax
import jax.numpy as jnp
from jax import lax
from jax.experimental import pallas as pl
from jax.experimental.pallas import tpu as pltpu
from jax.experimental.pallas import tpu_sc as plsc

N = 10000
E = 160000
T = 64000
H = 128
NR = 6
NB = 8
NBLK = 2
SBF_DIM = 42

NC = 2
NSUB = 16
NW = NC * NSUB

BE = 640
BT = 512

_silu = jax.nn.silu


def _dot(a, b):
  return jnp.dot(a, b, preferred_element_type=jnp.float32)


def _node_embed_body(x_ref, f1w, f1b, bng, bnb, f2w, f2b, w1, w2, a_ref):
  x = x_ref[...]
  h1 = _dot(x, f1w[...]) + f1b[...]
  mu = jnp.mean(h1, axis=0, keepdims=True)
  var = jnp.mean((h1 - mu) ** 2, axis=0, keepdims=True)
  h1 = (h1 - mu) / jnp.sqrt(var + 1e-5) * bng[...] + bnb[...]
  h1 = jnp.maximum(h1, 0.0)
  h = jnp.maximum(_dot(h1, f2w[...]) + f2b[...], 0.0)
  a_ref[:N, :] = _dot(h, w1[...])
  a_ref[N:, :] = _dot(h, w2[...])


def _node_embed(x, p):
  return pl.pallas_call(
      _node_embed_body,
      out_shape=jax.ShapeDtypeStruct((2 * N, H), jnp.float32),
  )(x, p["f1"]["w"], p["f1"]["b"].reshape(1, -1),
    p["bng"].reshape(1, -1), p["bnb"].reshape(1, -1),
    p["f2"]["w"], p["f2"]["b"].reshape(1, -1),
    p["lin"]["w"][:H], p["lin"]["w"][H:2 * H])


def _edge_embed_body(s1_ref, s2_ref, rbf_ref, wr, br, w3, bl, wro,
                     m_ref, t0_ref):
  rbf = rbf_ref[...]
  r = _silu(_dot(rbf, wr[...]) + br[...])
  m = _silu(s1_ref[...] + s2_ref[...] + _dot(r, w3[...]) + bl[...])
  m_ref[...] = m
  t0_ref[...] = _dot(rbf, wro[...]) * m


def _edge_embed(s12, rbf, p, wro):
  eb = lambda i: (i, 0)
  full = lambda i: (0, 0)
  return pl.pallas_call(
      _edge_embed_body,
      grid=(E // BE,),
      in_specs=[
          pl.BlockSpec((BE, H), eb),
          pl.BlockSpec((BE, H), lambda i: (i + E // BE, 0)),
          pl.BlockSpec((BE, NR), eb),
          pl.BlockSpec((NR, H), full),
          pl.BlockSpec((1, H), full),
          pl.BlockSpec((H, H), full),
          pl.BlockSpec((1, H), full),
          pl.BlockSpec((NR, H), full),
      ],
      out_specs=(pl.BlockSpec((BE, H), eb), pl.BlockSpec((BE, H), eb)),
      out_shape=(jax.ShapeDtypeStruct((E, H), jnp.float32),
                 jax.ShapeDtypeStruct((E, H), jnp.float32)),
  )(s12, s12, rbf, p["rbf"]["w"], p["rbf"]["b"].reshape(1, -1),
    p["lin"]["w"][2 * H:], p["lin"]["b"].reshape(1, -1), wro)


def _out_node_body(np_ref, l1w, l1b, l2w, l2b, l3w, l3b, wout, out_ref):
  node = np_ref[...]
  node = _silu(_dot(node, l1w[...]) + l1b[...])
  node = _silu(_dot(node, l2w[...]) + l2b[...])
  node = _silu(_dot(node, l3w[...]) + l3b[...])
  tot = jnp.sum(node, axis=0, keepdims=True)
  out_ref[...] = _dot(tot, wout[...])


def _out_node(node, p):
  ls = p["lins"]
  return pl.pallas_call(
      _out_node_body,
      out_shape=jax.ShapeDtypeStruct((1, 1), jnp.float32),
  )(node,
    ls[0]["w"], ls[0]["b"].reshape(1, -1),
    ls[1]["w"], ls[1]["b"].reshape(1, -1),
    ls[2]["w"], ls[2]["b"].reshape(1, -1),
    p["out"])


def _inter1_body(m_ref, rbf_ref, wkj, bkj, wji, bji, wr, xji_ref, xkjr_ref):
  m = m_ref[...]
  xji_ref[...] = _silu(_dot(m, wji[...]) + bji[...])
  rbfp = _dot(rbf_ref[...], wr[...])
  xkjr_ref[...] = _silu(_dot(m, wkj[...]) + bkj[...]) * rbfp


def _inter1(m, rbf, p):
  eb = lambda i: (i, 0)
  full = lambda i: (0, 0)
  return pl.pallas_call(
      _inter1_body,
      grid=(E // BE,),
      in_specs=[
          pl.BlockSpec((BE, H), eb),
          pl.BlockSpec((BE, NR), eb),
          pl.BlockSpec((H, H), full),
          pl.BlockSpec((1, H), full),
          pl.BlockSpec((H, H), full),
          pl.BlockSpec((1, H), full),
          pl.BlockSpec((NR, H), full),
      ],
      out_specs=(pl.BlockSpec((BE, H), eb), pl.BlockSpec((BE, H), eb)),
      out_shape=(jax.ShapeDtypeStruct((E, H), jnp.float32),
                 jax.ShapeDtypeStruct((E, H), jnp.float32)),
  )(m, rbf, p["kj"]["w"], p["kj"]["b"].reshape(1, -1),
    p["ji"]["w"], p["ji"]["b"].reshape(1, -1), p["rbf"]["w"])


def _inter2_body(xg_ref, sbf_ref, ws, wt, t_ref):
  sp = _dot(sbf_ref[...], ws[...])
  xg = xg_ref[...]
  acc = sp[:, 0:1] * _dot(xg, wt[0])
  for b in range(1, NB):
    acc = acc + sp[:, b:b + 1] * _dot(xg, wt[b])
  t_ref[...] = acc


def _inter2(xg, sbf, p):
  wt = jnp.transpose(p["W"], (1, 2, 0))
  tb = lambda i: (i, 0)
  return pl.pallas_call(
      _inter2_body,
      grid=(T // BT,),
      in_specs=[
          pl.BlockSpec((BT, H), tb),
          pl.BlockSpec((BT, SBF_DIM), tb),
          pl.BlockSpec((SBF_DIM, NB), lambda i: (0, 0)),
          pl.BlockSpec((NB, H, H), lambda i: (0, 0, 0)),
      ],
      out_specs=pl.BlockSpec((BT, H), tb),
      out_shape=jax.ShapeDtypeStruct((T, H), jnp.float32),
  )(xg, sbf, p["sbf"]["w"], wt)


def _inter3_body(xji_ref, agg_ref, m_ref, rbf_ref,
                 b1w, b1b, b2w, b2b, lw, lb,
                 a1w, a1b, a2w, a2b, a3w, a3b, a4w, a4b, wro,
                 mo_ref, tn_ref):
  h = xji_ref[...] + agg_ref[...]
  h = h + _silu(_dot(_silu(_dot(h, b1w[...]) + b1b[...]), b2w[...]) + b2b[...])
  h = _silu(_dot(h, lw[...]) + lb[...]) + m_ref[...]
  h = h + _silu(_dot(_silu(_dot(h, a1w[...]) + a1b[...]), a2w[...]) + a2b[...])
  h = h + _silu(_dot(_silu(_dot(h, a3w[...]) + a3b[...]), a4w[...]) + a4b[...])
  mo_ref[...] = h
  tn_ref[...] = _dot(rbf_ref[...], wro[...]) * h


def _inter3(xji, agg, m, rbf, p, wro):
  eb = lambda i: (i, 0)
  full = lambda i: (0, 0)
  wspec = pl.BlockSpec((H, H), full)
  bspec = pl.BlockSpec((1, H), full)
  bef = p["before"][0]
  af0, af1 = p["after"][0], p["after"][1]
  return pl.pallas_call(
      _inter3_body,
      grid=(E // BE,),
      in_specs=[
          pl.BlockSpec((BE, H), eb),
          pl.BlockSpec((BE, H), eb),
          pl.BlockSpec((BE, H), eb),
          pl.BlockSpec((BE, NR), eb),
          wspec, bspec, wspec, bspec, wspec, bspec,
          wspec, bspec, wspec, bspec, wspec, bspec, wspec, bspec,
          pl.BlockSpec((NR, H), full),
      ],
      out_specs=(pl.BlockSpec((BE, H), eb), pl.BlockSpec((BE, H), eb)),
      out_shape=(jax.ShapeDtypeStruct((E, H), jnp.float32),
                 jax.ShapeDtypeStruct((E, H), jnp.float32)),
  )(xji, agg, m, rbf,
    bef["l1"]["w"], bef["l1"]["b"].reshape(1, -1),
    bef["l2"]["w"], bef["l2"]["b"].reshape(1, -1),
    p["lin"]["w"], p["lin"]["b"].reshape(1, -1),
    af0["l1"]["w"], af0["l1"]["b"].reshape(1, -1),
    af0["l2"]["w"], af0["l2"]["b"].reshape(1, -1),
    af1["l1"]["w"], af1["l1"]["b"].reshape(1, -1),
    af1["l2"]["w"], af1["l2"]["b"].reshape(1, -1),
    wro)


@functools.lru_cache(maxsize=None)
def _sc_mesh():
  return plsc.VectorSubcoreMesh(core_axis_name="c", subcore_axis_name="s")


@functools.lru_cache(maxsize=None)
def _make_gather(m_rows, chunk, kq):
  per_tile = m_rows // NW
  n_chunks = per_tile // chunk
  assert n_chunks % kq == 0

  @functools.partial(
      pl.kernel,
      out_type=jax.ShapeDtypeStruct((m_rows, H), jnp.float32),
      mesh=_sc_mesh(),
      scratch_types=[
          pltpu.VMEM((per_tile,), jnp.int32),
          pltpu.VMEM((kq, chunk, H), jnp.float32),
          pltpu.SemaphoreType.DMA,
          pltpu.SemaphoreType.DMA,
      ],
  )
  def k(table_hbm, idx_hbm, out_hbm, idx_all, rows, sem_g, sem_w):
    wid = lax.axis_index("s") * NC + lax.axis_index("c")
    base = wid * per_tile
    pltpu.sync_copy(idx_hbm.at[pl.ds(base, per_tile)], idx_all)

    def group(g, carry):
      gd = []
      for b in range(kq):
        off = (g * kq + b) * chunk
        gd.append(pltpu.async_copy(
            table_hbm.at[idx_all.at[pl.ds(off, chunk)]], rows.at[b], sem_g))
      for d in gd:
        d.wait()
      wd = []
      for b in range(kq):
        off = (g * kq + b) * chunk
        wd.append(pltpu.async_copy(
            rows.at[b], out_hbm.at[pl.ds(base + off, chunk)], sem_w))
      for d in wd:
        d.wait()
      return carry

    lax.fori_loop(0, n_chunks // kq, group, 0)

  return k


@functools.lru_cache(maxsize=None)
def _make_scatter(m_rows, win, wpc, zb, chunk, kb):
  per_tile = m_rows // NSUB
  n_groups = per_tile // 16
  cap = per_tile + chunk + 16
  wt_count = NSUB if (win // NSUB) % 8 == 0 else NSUB // 2
  rpt = win // wt_count
  n_zero = rpt // zb
  assert rpt % zb == 0 and per_tile % 16 == 0 and chunk % 16 == 0

  @functools.partial(
      pl.kernel,
      out_type=jax.ShapeDtypeStruct((2 * wpc * win, H), jnp.float32),
      mesh=_sc_mesh(),
      compiler_params=pltpu.CompilerParams(needs_layout_passes=False),
      scratch_types=[
          pltpu.VMEM_SHARED((win + 8, H), jnp.float32),
          pltpu.VMEM((per_tile,), jnp.int32),
          pltpu.VMEM((cap,), jnp.int32),
          pltpu.VMEM((cap,), jnp.int32),
          pltpu.VMEM((chunk,), jnp.int32),
          pltpu.VMEM((kb, chunk, H), jnp.float32),
          pltpu.SemaphoreType.DMA,
          pltpu.SemaphoreType.DMA,
      ],
  )
  def k(src_hbm, idx_hbm, zeros_hbm, out_hbm, shared, idx_all,
        dest_list, tid_list, dest_v, bufs, sem_g, sem_z):
    cid = lax.axis_index("c")
    sid = lax.axis_index("s")
    src_base = sid * per_tile
    pltpu.sync_copy(idx_hbm.at[pl.ds(src_base, per_tile)], idx_all)
    iota16 = lax.iota(jnp.int32, 16)

    def window(w, carry):
      lo = (cid * wpc + w) * win
      out_base = (cid * wpc + w) * win

      @pl.when(sid < wt_count)
      def _():
        for z in range(n_zero):
          pltpu.async_copy(
              zeros_hbm, shared.at[pl.ds(sid * rpt + z * zb, zb)], sem_z)

      def grp(g, cnt):
        v = idx_all[pl.ds(g * 16, 16)] - lo
        ok = ((v >= 0) & (v < win)).astype(jnp.int32)
        pre = plsc.cumsum(ok)
        pos = jnp.where(ok > 0, cnt + pre - ok, (cap - 16) + iota16)
        plsc.store_scatter(dest_list, [pos], v)
        plsc.store_scatter(tid_list, [pos], iota16 + (src_base + g * 16))
        return cnt + jnp.sum(ok)

      cnt = lax.fori_loop(0, n_groups, grp, 0)

      @pl.when(sid < wt_count)
      def _():
        for z in range(n_zero):
          pltpu.make_async_copy(
              zeros_hbm, shared.at[pl.ds(sid * rpt + z * zb, zb)],
              sem_z).wait()

      plsc.subcore_barrier()
      for kp in range(chunk // 16):
        pos = cnt + kp * 16 + iota16
        plsc.store_scatter(dest_list, [pos], jnp.full((16,), win, jnp.int32))
        plsc.store_scatter(tid_list, [pos],
                           jnp.full((16,), src_base, jnp.int32))

      n_slots = (per_tile + chunk - 1) // chunk
      n_cgroups = (n_slots + kb - 1) // kb

      def cgroup(g2, cc):
        for b in range(kb):
          c = g2 * kb + b
          if True:
            @pl.when(c * chunk < cnt)
            def _(c=c, b=b):
              off = c * chunk
              pltpu.async_copy(
                  src_hbm.at[tid_list.at[pl.ds(off, chunk)]],
                  bufs.at[b], sem_g)

        for b in range(kb):
          c = g2 * kb + b
          if True:
            @pl.when(c * chunk < cnt)
            def _(c=c, b=b):
              pltpu.make_async_copy(
                  src_hbm.at[pl.ds(0, chunk)], bufs.at[b], sem_g).wait()

        for b in range(kb):
          c = g2 * kb + b
          if True:
            @pl.when(c * chunk < cnt)
            def _(c=c, b=b):
              off = c * chunk
              for q in range(chunk // 16):
                dest_v[pl.ds(q * 16, 16)] = dest_list[pl.ds(off + q * 16, 16)]
              pltpu.sync_copy(bufs.at[b], shared.at[dest_v], add=True)

        return cc

      lax.fori_loop(0, n_cgroups, cgroup, 0)
      plsc.subcore_barrier()

      @pl.when(sid < wt_count)
      def _():
        pltpu.sync_copy(shared.at[pl.ds(sid * rpt, rpt)],
                        out_hbm.at[pl.ds(out_base + sid * rpt, rpt)])

      plsc.subcore_barrier()
      return carry

    lax.fori_loop(0, wpc, window, 0)

  return k


def _zeros_buf(zb):
  return jnp.zeros((zb, H), jnp.float32)


def _sc_gather(table, idx, m_rows, chunk, kq):
  return _make_gather(m_rows, chunk, kq)(table, idx)


def _segsum_edges(t, idx_ji):
  return _make_scatter(T, 8000, 10, 125, 80, 5)(t, idx_ji, _zeros_buf(125))


def _segsum_nodes(t, i):
  out = _make_scatter(E, 5120, 1, 160, 80, 5)(t, i, _zeros_buf(160))
  return out[:N]


def kernel(x, rbf, sbf, i, j, idx_kj, idx_ji, params):
  i = i.astype(jnp.int32)
  j = j.astype(jnp.int32)
  idx_kj = idx_kj.astype(jnp.int32)
  idx_ji = idx_ji.astype(jnp.int32)

  a12 = _node_embed(x, params["emb"])
  ij = jnp.concatenate([i, j + N])
  s12 = _sc_gather(a12, ij, 2 * E, 40, 10)
  m, t_out = _edge_embed(s12, rbf, params["emb"],
                         params["out"][0]["rbf"]["w"])
  total = _out_node(_segsum_nodes(t_out, i), params["out"][0])
  for b in range(NBLK):
    pb = params["inter"][b]
    xji, xkjr = _inter1(m, rbf, pb)
    xg = _sc_gather(xkjr, idx_kj, T, 80, 5)
    t = _inter2(xg, sbf, pb)
    agg = _segsum_edges(t, idx_ji)
    m, t_out = _inter3(xji, agg, m, rbf, pb,
                       params["out"][b + 1]["rbf"]["w"])
    total = total + _out_node(_segsum_nodes(t_out, i), params["out"][b + 1])
  return total.reshape((1,))


# speedup vs baseline: 2.1951x; 1.2400x over previous
import functools

import jax
import jax.numpy as jnp
from jax import lax
from jax.experimental import pallas as pl
from jax.experimental.pallas import tpu as pltpu
from jax.experimental.pallas import tpu_sc as plsc

N = 10000
E = 160000
T = 64000
H = 128
NR = 6
NB = 8
NBLK = 2
SBF_DIM = 42

NC = 2
NSUB = 16
NW = NC * NSUB

BE = 1280
BT = 1000

_silu = jax.nn.silu


def _dot(a, b):
  return jnp.dot(a, b, preferred_element_type=jnp.float32)


def _node_embed_body(x_ref, f1w, f1b, bng, bnb, f2w, f2b, w1, w2, a_ref):
  x = x_ref[...]
  h1 = _dot(x, f1w[...]) + f1b[...]
  mu = jnp.mean(h1, axis=0, keepdims=True)
  var = jnp.mean((h1 - mu) ** 2, axis=0, keepdims=True)
  h1 = (h1 - mu) / jnp.sqrt(var + 1e-5) * bng[...] + bnb[...]
  h1 = jnp.maximum(h1, 0.0)
  h = jnp.maximum(_dot(h1, f2w[...]) + f2b[...], 0.0)
  a_ref[:N, :] = _dot(h, w1[...])
  a_ref[N:, :] = _dot(h, w2[...])


def _node_embed(x, p):
  return pl.pallas_call(
      _node_embed_body,
      out_shape=jax.ShapeDtypeStruct((2 * N, H), jnp.float32),
  )(x, p["f1"]["w"], p["f1"]["b"].reshape(1, -1),
    p["bng"].reshape(1, -1), p["bnb"].reshape(1, -1),
    p["f2"]["w"], p["f2"]["b"].reshape(1, -1),
    p["lin"]["w"][:H], p["lin"]["w"][H:2 * H])


def _edge_embed_body(s1_ref, s2_ref, rbf_ref, wr, br, w3, bl, wro,
                     m_ref, t0_ref):
  rbf = rbf_ref[...]
  r = _silu(_dot(rbf, wr[...]) + br[...])
  m = _silu(s1_ref[...] + s2_ref[...] + _dot(r, w3[...]) + bl[...])
  m_ref[...] = m
  t0_ref[...] = _dot(rbf, wro[...]) * m


def _edge_embed(s12, rbf, p, wro):
  eb = lambda i: (i, 0)
  full = lambda i: (0, 0)
  return pl.pallas_call(
      _edge_embed_body,
      grid=(E // BE,),
      in_specs=[
          pl.BlockSpec((BE, H), eb),
          pl.BlockSpec((BE, H), lambda i: (i + E // BE, 0)),
          pl.BlockSpec((BE, NR), eb),
          pl.BlockSpec((NR, H), full),
          pl.BlockSpec((1, H), full),
          pl.BlockSpec((H, H), full),
          pl.BlockSpec((1, H), full),
          pl.BlockSpec((NR, H), full),
      ],
      out_specs=(pl.BlockSpec((BE, H), eb), pl.BlockSpec((BE, H), eb)),
      out_shape=(jax.ShapeDtypeStruct((E, H), jnp.float32),
                 jax.ShapeDtypeStruct((E, H), jnp.float32)),
  )(s12, s12, rbf, p["rbf"]["w"], p["rbf"]["b"].reshape(1, -1),
    p["lin"]["w"][2 * H:], p["lin"]["b"].reshape(1, -1), wro)


def _out_node_body(np_ref, l1w, l1b, l2w, l2b, l3w, l3b, wout, out_ref):
  node = np_ref[...]
  node = _silu(_dot(node, l1w[...]) + l1b[...])
  node = _silu(_dot(node, l2w[...]) + l2b[...])
  node = _silu(_dot(node, l3w[...]) + l3b[...])
  tot = jnp.sum(node, axis=0, keepdims=True)
  out_ref[...] = _dot(tot, wout[...])


def _out_node(node, p):
  ls = p["lins"]
  return pl.pallas_call(
      _out_node_body,
      out_shape=jax.ShapeDtypeStruct((1, 1), jnp.float32),
  )(node,
    ls[0]["w"], ls[0]["b"].reshape(1, -1),
    ls[1]["w"], ls[1]["b"].reshape(1, -1),
    ls[2]["w"], ls[2]["b"].reshape(1, -1),
    p["out"])


def _inter1_body(m_ref, rbf_ref, wkj, bkj, wji, bji, wr, xji_ref, xkjr_ref):
  m = m_ref[...]
  xji_ref[...] = _silu(_dot(m, wji[...]) + bji[...])
  rbfp = _dot(rbf_ref[...], wr[...])
  xkjr_ref[...] = _silu(_dot(m, wkj[...]) + bkj[...]) * rbfp


def _inter1(m, rbf, p):
  eb = lambda i: (i, 0)
  full = lambda i: (0, 0)
  return pl.pallas_call(
      _inter1_body,
      grid=(E // BE,),
      in_specs=[
          pl.BlockSpec((BE, H), eb),
          pl.BlockSpec((BE, NR), eb),
          pl.BlockSpec((H, H), full),
          pl.BlockSpec((1, H), full),
          pl.BlockSpec((H, H), full),
          pl.BlockSpec((1, H), full),
          pl.BlockSpec((NR, H), full),
      ],
      out_specs=(pl.BlockSpec((BE, H), eb), pl.BlockSpec((BE, H), eb)),
      out_shape=(jax.ShapeDtypeStruct((E, H), jnp.float32),
                 jax.ShapeDtypeStruct((E, H), jnp.float32)),
  )(m, rbf, p["kj"]["w"], p["kj"]["b"].reshape(1, -1),
    p["ji"]["w"], p["ji"]["b"].reshape(1, -1), p["rbf"]["w"])


def _inter2_body(xg_ref, sbf_ref, ws, wt, t_ref):
  sp = _dot(sbf_ref[...], ws[...])
  xg = xg_ref[...]
  acc = sp[:, 0:1] * _dot(xg, wt[0])
  for b in range(1, NB):
    acc = acc + sp[:, b:b + 1] * _dot(xg, wt[b])
  t_ref[...] = acc


def _inter2(xg, sbf, p):
  wt = jnp.transpose(p["W"], (1, 2, 0))
  tb = lambda i: (i, 0)
  return pl.pallas_call(
      _inter2_body,
      grid=(T // BT,),
      in_specs=[
          pl.BlockSpec((BT, H), tb),
          pl.BlockSpec((BT, SBF_DIM), tb),
          pl.BlockSpec((SBF_DIM, NB), lambda i: (0, 0)),
          pl.BlockSpec((NB, H, H), lambda i: (0, 0, 0)),
      ],
      out_specs=pl.BlockSpec((BT, H), tb),
      out_shape=jax.ShapeDtypeStruct((T, H), jnp.float32),
  )(xg, sbf, p["sbf"]["w"], wt)


def _inter3_body(xji_ref, agg_ref, m_ref, rbf_ref,
                 b1w, b1b, b2w, b2b, lw, lb,
                 a1w, a1b, a2w, a2b, a3w, a3b, a4w, a4b, wro,
                 mo_ref, tn_ref):
  h = xji_ref[...] + agg_ref[...]
  h = h + _silu(_dot(_silu(_dot(h, b1w[...]) + b1b[...]), b2w[...]) + b2b[...])
  h = _silu(_dot(h, lw[...]) + lb[...]) + m_ref[...]
  h = h + _silu(_dot(_silu(_dot(h, a1w[...]) + a1b[...]), a2w[...]) + a2b[...])
  h = h + _silu(_dot(_silu(_dot(h, a3w[...]) + a3b[...]), a4w[...]) + a4b[...])
  mo_ref[...] = h
  tn_ref[...] = _dot(rbf_ref[...], wro[...]) * h


def _inter3(xji, agg, m, rbf, p, wro):
  eb = lambda i: (i, 0)
  full = lambda i: (0, 0)
  wspec = pl.BlockSpec((H, H), full)
  bspec = pl.BlockSpec((1, H), full)
  bef = p["before"][0]
  af0, af1 = p["after"][0], p["after"][1]
  return pl.pallas_call(
      _inter3_body,
      grid=(E // BE,),
      in_specs=[
          pl.BlockSpec((BE, H), eb),
          pl.BlockSpec((BE, H), eb),
          pl.BlockSpec((BE, H), eb),
          pl.BlockSpec((BE, NR), eb),
          wspec, bspec, wspec, bspec, wspec, bspec,
          wspec, bspec, wspec, bspec, wspec, bspec, wspec, bspec,
          pl.BlockSpec((NR, H), full),
      ],
      out_specs=(pl.BlockSpec((BE, H), eb), pl.BlockSpec((BE, H), eb)),
      out_shape=(jax.ShapeDtypeStruct((E, H), jnp.float32),
                 jax.ShapeDtypeStruct((E, H), jnp.float32)),
  )(xji, agg, m, rbf,
    bef["l1"]["w"], bef["l1"]["b"].reshape(1, -1),
    bef["l2"]["w"], bef["l2"]["b"].reshape(1, -1),
    p["lin"]["w"], p["lin"]["b"].reshape(1, -1),
    af0["l1"]["w"], af0["l1"]["b"].reshape(1, -1),
    af0["l2"]["w"], af0["l2"]["b"].reshape(1, -1),
    af1["l1"]["w"], af1["l1"]["b"].reshape(1, -1),
    af1["l2"]["w"], af1["l2"]["b"].reshape(1, -1),
    wro)


@functools.lru_cache(maxsize=None)
def _sc_mesh():
  return plsc.VectorSubcoreMesh(core_axis_name="c", subcore_axis_name="s")


@functools.lru_cache(maxsize=None)
def _make_gather(m_rows, chunk, kq):
  per_tile = m_rows // NW
  n_chunks = per_tile // chunk
  assert n_chunks % kq == 0

  @functools.partial(
      pl.kernel,
      out_type=jax.ShapeDtypeStruct((m_rows, H), jnp.float32),
      mesh=_sc_mesh(),
      scratch_types=[
          pltpu.VMEM((per_tile,), jnp.int32),
          pltpu.VMEM((kq, chunk, H), jnp.float32),
          pltpu.SemaphoreType.DMA,
          pltpu.SemaphoreType.DMA,
      ],
  )
  def k(table_hbm, idx_hbm, out_hbm, idx_all, rows, sem_g, sem_w):
    wid = lax.axis_index("s") * NC + lax.axis_index("c")
    base = wid * per_tile
    pltpu.sync_copy(idx_hbm.at[pl.ds(base, per_tile)], idx_all)

    def group(g, carry):
      gd = []
      for b in range(kq):
        off = (g * kq + b) * chunk
        gd.append(pltpu.async_copy(
            table_hbm.at[idx_all.at[pl.ds(off, chunk)]], rows.at[b], sem_g))
      for d in gd:
        d.wait()
      wd = []
      for b in range(kq):
        off = (g * kq + b) * chunk
        wd.append(pltpu.async_copy(
            rows.at[b], out_hbm.at[pl.ds(base + off, chunk)], sem_w))
      for d in wd:
        d.wait()
      return carry

    lax.fori_loop(0, n_chunks // kq, group, 0)

  return k


@functools.lru_cache(maxsize=None)
def _make_scatter(m_rows, win, wpc, zb, chunk, kb):
  per_tile = m_rows // NSUB
  n_groups = per_tile // 16
  cap = per_tile + chunk + 16
  wt_count = NSUB if (win // NSUB) % 8 == 0 else NSUB // 2
  rpt = win // wt_count
  n_zero = rpt // zb
  assert rpt % zb == 0 and per_tile % 16 == 0 and chunk % 16 == 0

  @functools.partial(
      pl.kernel,
      out_type=jax.ShapeDtypeStruct((2 * wpc * win, H), jnp.float32),
      mesh=_sc_mesh(),
      compiler_params=pltpu.CompilerParams(needs_layout_passes=False),
      scratch_types=[
          pltpu.VMEM_SHARED((win + 8, H), jnp.float32),
          pltpu.VMEM((per_tile,), jnp.int32),
          pltpu.VMEM((cap,), jnp.int32),
          pltpu.VMEM((cap,), jnp.int32),
          pltpu.VMEM((chunk,), jnp.int32),
          pltpu.VMEM((kb, chunk, H), jnp.float32),
          pltpu.SemaphoreType.DMA,
          pltpu.SemaphoreType.DMA,
      ],
  )
  def k(src_hbm, idx_hbm, zeros_hbm, out_hbm, shared, idx_all,
        dest_list, tid_list, dest_v, bufs, sem_g, sem_z):
    cid = lax.axis_index("c")
    sid = lax.axis_index("s")
    src_base = sid * per_tile
    pltpu.sync_copy(idx_hbm.at[pl.ds(src_base, per_tile)], idx_all)
    iota16 = lax.iota(jnp.int32, 16)

    def window(w, carry):
      lo = (cid * wpc + w) * win
      out_base = (cid * wpc + w) * win

      @pl.when(sid < wt_count)
      def _():
        for z in range(n_zero):
          pltpu.async_copy(
              zeros_hbm, shared.at[pl.ds(sid * rpt + z * zb, zb)], sem_z)

      def grp(g, cnt):
        v = idx_all[pl.ds(g * 16, 16)] - lo
        ok = ((v >= 0) & (v < win)).astype(jnp.int32)
        pre = plsc.cumsum(ok)
        pos = jnp.where(ok > 0, cnt + pre - ok, (cap - 16) + iota16)
        plsc.store_scatter(dest_list, [pos], v)
        plsc.store_scatter(tid_list, [pos], iota16 + (src_base + g * 16))
        return cnt + jnp.sum(ok)

      cnt = lax.fori_loop(0, n_groups, grp, 0)

      @pl.when(sid < wt_count)
      def _():
        for z in range(n_zero):
          pltpu.make_async_copy(
              zeros_hbm, shared.at[pl.ds(sid * rpt + z * zb, zb)],
              sem_z).wait()

      plsc.subcore_barrier()
      for kp in range(chunk // 16):
        pos = cnt + kp * 16 + iota16
        plsc.store_scatter(dest_list, [pos], jnp.full((16,), win, jnp.int32))
        plsc.store_scatter(tid_list, [pos],
                           jnp.full((16,), src_base, jnp.int32))

      n_slots = (per_tile + chunk - 1) // chunk
      n_cgroups = (n_slots + kb - 1) // kb

      def cgroup(g2, cc):
        for b in range(kb):
          c = g2 * kb + b
          if True:
            @pl.when(c * chunk < cnt)
            def _(c=c, b=b):
              off = c * chunk
              pltpu.async_copy(
                  src_hbm.at[tid_list.at[pl.ds(off, chunk)]],
                  bufs.at[b], sem_g)

        for b in range(kb):
          c = g2 * kb + b
          if True:
            @pl.when(c * chunk < cnt)
            def _(c=c, b=b):
              pltpu.make_async_copy(
                  src_hbm.at[pl.ds(0, chunk)], bufs.at[b], sem_g).wait()

        for b in range(kb):
          c = g2 * kb + b
          if True:
            @pl.when(c * chunk < cnt)
            def _(c=c, b=b):
              off = c * chunk
              for q in range(chunk // 16):
                dest_v[pl.ds(q * 16, 16)] = dest_list[pl.ds(off + q * 16, 16)]
              pltpu.sync_copy(bufs.at[b], shared.at[dest_v], add=True)

        return cc

      lax.fori_loop(0, n_cgroups, cgroup, 0)
      plsc.subcore_barrier()

      @pl.when(sid < wt_count)
      def _():
        pltpu.sync_copy(shared.at[pl.ds(sid * rpt, rpt)],
                        out_hbm.at[pl.ds(out_base + sid * rpt, rpt)])

      plsc.subcore_barrier()
      return carry

    lax.fori_loop(0, wpc, window, 0)

  return k


def _zeros_buf(zb):
  return jnp.zeros((zb, H), jnp.float32)


def _sc_gather(table, idx, m_rows, chunk, kq):
  return _make_gather(m_rows, chunk, kq)(table, idx)


def _segsum_edges(t, idx_ji):
  return _make_scatter(T, 8000, 10, 125, 80, 5)(t, idx_ji, _zeros_buf(125))


def _segsum_nodes(t, i):
  out = _make_scatter(E, 5120, 1, 160, 80, 5)(t, i, _zeros_buf(160))
  return out[:N]


def kernel(x, rbf, sbf, i, j, idx_kj, idx_ji, params):
  i = i.astype(jnp.int32)
  j = j.astype(jnp.int32)
  idx_kj = idx_kj.astype(jnp.int32)
  idx_ji = idx_ji.astype(jnp.int32)

  a12 = _node_embed(x, params["emb"])
  ij = jnp.concatenate([i, j + N])
  s12 = _sc_gather(a12, ij, 2 * E, 40, 10)
  m, t_out = _edge_embed(s12, rbf, params["emb"],
                         params["out"][0]["rbf"]["w"])
  total = jnp.zeros((1, 1), jnp.float32)
  for b in range(NBLK):
    pb = params["inter"][b]
    xji, xkjr = _inter1(m, rbf, pb)
    xg = _sc_gather(xkjr, idx_kj, T, 80, 5)
    total = total + _out_node(_segsum_nodes(t_out, i), params["out"][b])
    t = _inter2(xg, sbf, pb)
    agg = _segsum_edges(t, idx_ji)
    m, t_out = _inter3(xji, agg, m, rbf, pb,
                       params["out"][b + 1]["rbf"]["w"])
  total = total + _out_node(_segsum_nodes(t_out, i), params["out"][NBLK])
  return total.reshape((1,))


# speedup vs baseline: 2.2017x; 1.0030x over previous
import functools

import jax
import jax.numpy as jnp
from jax import lax
from jax.experimental import pallas as pl
from jax.experimental.pallas import tpu as pltpu
from jax.experimental.pallas import tpu_sc as plsc

N = 10000
E = 160000
T = 64000
H = 128
NR = 6
NB = 8
NBLK = 2
SBF_DIM = 42

NC = 2
NSUB = 16
NW = NC * NSUB

BE = 1280
BT = 1000

_silu = jax.nn.silu


def _dot(a, b):
  return jnp.dot(a, b, preferred_element_type=jnp.float32)


def _node_embed_body(x_ref, f1w, f1b, bng, bnb, f2w, f2b, w1, w2, a_ref):
  x = x_ref[...]
  h1 = _dot(x, f1w[...]) + f1b[...]
  mu = jnp.mean(h1, axis=0, keepdims=True)
  var = jnp.mean((h1 - mu) ** 2, axis=0, keepdims=True)
  h1 = (h1 - mu) / jnp.sqrt(var + 1e-5) * bng[...] + bnb[...]
  h1 = jnp.maximum(h1, 0.0)
  h = jnp.maximum(_dot(h1, f2w[...]) + f2b[...], 0.0)
  a_ref[:N, :] = _dot(h, w1[...])
  a_ref[N:, :] = _dot(h, w2[...])


def _node_embed(x, p):
  return pl.pallas_call(
      _node_embed_body,
      out_shape=jax.ShapeDtypeStruct((2 * N, H), jnp.float32),
  )(x, p["f1"]["w"], p["f1"]["b"].reshape(1, -1),
    p["bng"].reshape(1, -1), p["bnb"].reshape(1, -1),
    p["f2"]["w"], p["f2"]["b"].reshape(1, -1),
    p["lin"]["w"][:H], p["lin"]["w"][H:2 * H])


def _edge_embed_body(s1_ref, s2_ref, rbf_ref, wr, br, w3, bl, wro,
                     m_ref, t0_ref):
  rbf = rbf_ref[...]
  r = _silu(_dot(rbf, wr[...]) + br[...])
  m = _silu(s1_ref[...] + s2_ref[...] + _dot(r, w3[...]) + bl[...])
  m_ref[...] = m
  t0_ref[...] = _dot(rbf, wro[...]) * m


def _edge_embed(s12, rbf, p, wro):
  eb = lambda i: (i, 0)
  full = lambda i: (0, 0)
  return pl.pallas_call(
      _edge_embed_body,
      grid=(E // BE,),
      in_specs=[
          pl.BlockSpec((BE, H), eb),
          pl.BlockSpec((BE, H), lambda i: (i + E // BE, 0)),
          pl.BlockSpec((BE, NR), eb),
          pl.BlockSpec((NR, H), full),
          pl.BlockSpec((1, H), full),
          pl.BlockSpec((H, H), full),
          pl.BlockSpec((1, H), full),
          pl.BlockSpec((NR, H), full),
      ],
      out_specs=(pl.BlockSpec((BE, H), eb), pl.BlockSpec((BE, H), eb)),
      out_shape=(jax.ShapeDtypeStruct((E, H), jnp.float32),
                 jax.ShapeDtypeStruct((E, H), jnp.float32)),
  )(s12, s12, rbf, p["rbf"]["w"], p["rbf"]["b"].reshape(1, -1),
    p["lin"]["w"][2 * H:], p["lin"]["b"].reshape(1, -1), wro)


def _out_node_body(np_ref, l1w, l1b, l2w, l2b, l3w, l3b, wout, out_ref):
  node = np_ref[...]
  node = _silu(_dot(node, l1w[...]) + l1b[...])
  node = _silu(_dot(node, l2w[...]) + l2b[...])
  node = _silu(_dot(node, l3w[...]) + l3b[...])
  tot = jnp.sum(node, axis=0, keepdims=True)
  out_ref[...] = _dot(tot, wout[...])


def _out_node(node, p):
  ls = p["lins"]
  return pl.pallas_call(
      _out_node_body,
      out_shape=jax.ShapeDtypeStruct((1, 1), jnp.float32),
  )(node,
    ls[0]["w"], ls[0]["b"].reshape(1, -1),
    ls[1]["w"], ls[1]["b"].reshape(1, -1),
    ls[2]["w"], ls[2]["b"].reshape(1, -1),
    p["out"])


def _inter1_body(m_ref, rbf_ref, wkj, bkj, wji, bji, wr, xji_ref, xkjr_ref):
  m = m_ref[...]
  xji_ref[...] = _silu(_dot(m, wji[...]) + bji[...])
  rbfp = _dot(rbf_ref[...], wr[...])
  xkjr_ref[...] = _silu(_dot(m, wkj[...]) + bkj[...]) * rbfp


def _inter1(m, rbf, p):
  eb = lambda i: (i, 0)
  full = lambda i: (0, 0)
  return pl.pallas_call(
      _inter1_body,
      grid=(E // BE,),
      in_specs=[
          pl.BlockSpec((BE, H), eb),
          pl.BlockSpec((BE, NR), eb),
          pl.BlockSpec((H, H), full),
          pl.BlockSpec((1, H), full),
          pl.BlockSpec((H, H), full),
          pl.BlockSpec((1, H), full),
          pl.BlockSpec((NR, H), full),
      ],
      out_specs=(pl.BlockSpec((BE, H), eb), pl.BlockSpec((BE, H), eb)),
      out_shape=(jax.ShapeDtypeStruct((E, H), jnp.float32),
                 jax.ShapeDtypeStruct((E, H), jnp.float32)),
  )(m, rbf, p["kj"]["w"], p["kj"]["b"].reshape(1, -1),
    p["ji"]["w"], p["ji"]["b"].reshape(1, -1), p["rbf"]["w"])


def _inter2_body(xg_ref, sbf_ref, ws, wt, t_ref):
  sp = _dot(sbf_ref[...], ws[...])
  xg = xg_ref[...]
  acc = sp[:, 0:1] * _dot(xg, wt[0])
  for b in range(1, NB):
    acc = acc + sp[:, b:b + 1] * _dot(xg, wt[b])
  t_ref[...] = acc


def _inter2(xg, sbf, p):
  wt = jnp.transpose(p["W"], (1, 2, 0))
  tb = lambda i: (i, 0)
  return pl.pallas_call(
      _inter2_body,
      grid=(T // BT,),
      in_specs=[
          pl.BlockSpec((BT, H), tb),
          pl.BlockSpec((BT, SBF_DIM), tb),
          pl.BlockSpec((SBF_DIM, NB), lambda i: (0, 0)),
          pl.BlockSpec((NB, H, H), lambda i: (0, 0, 0)),
      ],
      out_specs=pl.BlockSpec((BT, H), tb),
      out_shape=jax.ShapeDtypeStruct((T, H), jnp.float32),
  )(xg, sbf, p["sbf"]["w"], wt)


def _inter3_body(xji_ref, agg_ref, m_ref, rbf_ref,
                 b1w, b1b, b2w, b2b, lw, lb,
                 a1w, a1b, a2w, a2b, a3w, a3b, a4w, a4b, wro,
                 mo_ref, tn_ref):
  h = xji_ref[...] + agg_ref[...]
  h = h + _silu(_dot(_silu(_dot(h, b1w[...]) + b1b[...]), b2w[...]) + b2b[...])
  h = _silu(_dot(h, lw[...]) + lb[...]) + m_ref[...]
  h = h + _silu(_dot(_silu(_dot(h, a1w[...]) + a1b[...]), a2w[...]) + a2b[...])
  h = h + _silu(_dot(_silu(_dot(h, a3w[...]) + a3b[...]), a4w[...]) + a4b[...])
  mo_ref[...] = h
  tn_ref[...] = _dot(rbf_ref[...], wro[...]) * h


def _inter3(xji, agg, m, rbf, p, wro):
  eb = lambda i: (i, 0)
  full = lambda i: (0, 0)
  wspec = pl.BlockSpec((H, H), full)
  bspec = pl.BlockSpec((1, H), full)
  bef = p["before"][0]
  af0, af1 = p["after"][0], p["after"][1]
  return pl.pallas_call(
      _inter3_body,
      grid=(E // BE,),
      in_specs=[
          pl.BlockSpec((BE, H), eb),
          pl.BlockSpec((BE, H), eb),
          pl.BlockSpec((BE, H), eb),
          pl.BlockSpec((BE, NR), eb),
          wspec, bspec, wspec, bspec, wspec, bspec,
          wspec, bspec, wspec, bspec, wspec, bspec, wspec, bspec,
          pl.BlockSpec((NR, H), full),
      ],
      out_specs=(pl.BlockSpec((BE, H), eb), pl.BlockSpec((BE, H), eb)),
      out_shape=(jax.ShapeDtypeStruct((E, H), jnp.float32),
                 jax.ShapeDtypeStruct((E, H), jnp.float32)),
  )(xji, agg, m, rbf,
    bef["l1"]["w"], bef["l1"]["b"].reshape(1, -1),
    bef["l2"]["w"], bef["l2"]["b"].reshape(1, -1),
    p["lin"]["w"], p["lin"]["b"].reshape(1, -1),
    af0["l1"]["w"], af0["l1"]["b"].reshape(1, -1),
    af0["l2"]["w"], af0["l2"]["b"].reshape(1, -1),
    af1["l1"]["w"], af1["l1"]["b"].reshape(1, -1),
    af1["l2"]["w"], af1["l2"]["b"].reshape(1, -1),
    wro)


@functools.lru_cache(maxsize=None)
def _sc_mesh():
  return plsc.VectorSubcoreMesh(core_axis_name="c", subcore_axis_name="s")


@functools.lru_cache(maxsize=None)
def _make_gather(m_rows, chunk, kq):
  per_tile = m_rows // NW
  n_chunks = per_tile // chunk
  assert n_chunks % kq == 0

  @functools.partial(
      pl.kernel,
      out_type=jax.ShapeDtypeStruct((m_rows, H), jnp.float32),
      mesh=_sc_mesh(),
      scratch_types=[
          pltpu.VMEM((per_tile,), jnp.int32),
          pltpu.VMEM((kq, chunk, H), jnp.float32),
          pltpu.SemaphoreType.DMA,
          pltpu.SemaphoreType.DMA,
      ],
  )
  def k(table_hbm, idx_hbm, out_hbm, idx_all, rows, sem_g, sem_w):
    wid = lax.axis_index("s") * NC + lax.axis_index("c")
    base = wid * per_tile
    pltpu.sync_copy(idx_hbm.at[pl.ds(base, per_tile)], idx_all)

    def group(g, carry):
      gd = []
      for b in range(kq):
        off = (g * kq + b) * chunk
        gd.append(pltpu.async_copy(
            table_hbm.at[idx_all.at[pl.ds(off, chunk)]], rows.at[b], sem_g))
      for d in gd:
        d.wait()
      wd = []
      for b in range(kq):
        off = (g * kq + b) * chunk
        wd.append(pltpu.async_copy(
            rows.at[b], out_hbm.at[pl.ds(base + off, chunk)], sem_w))
      for d in wd:
        d.wait()
      return carry

    lax.fori_loop(0, n_chunks // kq, group, 0)

  return k


@functools.lru_cache(maxsize=None)
def _make_scatter(m_rows, win, wpc, zb, chunk, kb):
  per_tile = m_rows // NSUB
  n_groups = per_tile // 16
  cap = per_tile + chunk + 16
  wt_count = NSUB if (win // NSUB) % 8 == 0 else NSUB // 2
  rpt = win // wt_count
  n_zero = rpt // zb
  assert rpt % zb == 0 and per_tile % 16 == 0 and chunk % 16 == 0

  @functools.partial(
      pl.kernel,
      out_type=jax.ShapeDtypeStruct((2 * wpc * win, H), jnp.float32),
      mesh=_sc_mesh(),
      compiler_params=pltpu.CompilerParams(needs_layout_passes=False),
      scratch_types=[
          pltpu.VMEM_SHARED((win + 8, H), jnp.float32),
          pltpu.VMEM((per_tile,), jnp.int32),
          pltpu.VMEM((cap,), jnp.int32),
          pltpu.VMEM((cap,), jnp.int32),
          pltpu.VMEM((chunk,), jnp.int32),
          pltpu.VMEM((kb, chunk, H), jnp.float32),
          pltpu.SemaphoreType.DMA,
          pltpu.SemaphoreType.DMA,
      ],
  )
  def k(src_hbm, idx_hbm, zeros_hbm, out_hbm, shared, idx_all,
        dest_list, tid_list, dest_v, bufs, sem_g, sem_z):
    cid = lax.axis_index("c")
    sid = lax.axis_index("s")
    src_base = sid * per_tile
    pltpu.sync_copy(idx_hbm.at[pl.ds(src_base, per_tile)], idx_all)
    iota16 = lax.iota(jnp.int32, 16)

    def window(w, carry):
      lo = (cid * wpc + w) * win
      out_base = (cid * wpc + w) * win

      @pl.when(sid < wt_count)
      def _():
        for z in range(n_zero):
          pltpu.async_copy(
              zeros_hbm, shared.at[pl.ds(sid * rpt + z * zb, zb)], sem_z)

      def grp(g, cnt):
        v = idx_all[pl.ds(g * 16, 16)] - lo
        ok = ((v >= 0) & (v < win)).astype(jnp.int32)
        pre = plsc.cumsum(ok)
        pos = jnp.where(ok > 0, cnt + pre - ok, (cap - 16) + iota16)
        plsc.store_scatter(dest_list, [pos], v)
        plsc.store_scatter(tid_list, [pos], iota16 + (src_base + g * 16))
        return cnt + jnp.sum(ok)

      cnt = lax.fori_loop(0, n_groups, grp, 0)

      @pl.when(sid < wt_count)
      def _():
        for z in range(n_zero):
          pltpu.make_async_copy(
              zeros_hbm, shared.at[pl.ds(sid * rpt + z * zb, zb)],
              sem_z).wait()

      plsc.subcore_barrier()
      for kp in range(chunk // 16):
        pos = cnt + kp * 16 + iota16
        plsc.store_scatter(dest_list, [pos], jnp.full((16,), win, jnp.int32))
        plsc.store_scatter(tid_list, [pos],
                           jnp.full((16,), src_base, jnp.int32))

      n_slots = (per_tile + chunk - 1) // chunk
      n_cgroups = (n_slots + kb - 1) // kb

      def cgroup(g2, cc):
        for b in range(kb):
          c = g2 * kb + b
          if True:
            @pl.when(c * chunk < cnt)
            def _(c=c, b=b):
              off = c * chunk
              pltpu.async_copy(
                  src_hbm.at[tid_list.at[pl.ds(off, chunk)]],
                  bufs.at[b], sem_g)

        for b in range(kb):
          c = g2 * kb + b
          if True:
            @pl.when(c * chunk < cnt)
            def _(c=c, b=b):
              pltpu.make_async_copy(
                  src_hbm.at[pl.ds(0, chunk)], bufs.at[b], sem_g).wait()

        for b in range(kb):
          c = g2 * kb + b
          if True:
            @pl.when(c * chunk < cnt)
            def _(c=c, b=b):
              off = c * chunk
              for q in range(chunk // 16):
                dest_v[pl.ds(q * 16, 16)] = dest_list[pl.ds(off + q * 16, 16)]
              pltpu.sync_copy(bufs.at[b], shared.at[dest_v], add=True)

        return cc

      lax.fori_loop(0, n_cgroups, cgroup, 0)
      plsc.subcore_barrier()

      @pl.when(sid < wt_count)
      def _():
        pltpu.sync_copy(shared.at[pl.ds(sid * rpt, rpt)],
                        out_hbm.at[pl.ds(out_base + sid * rpt, rpt)])

      plsc.subcore_barrier()
      return carry

    lax.fori_loop(0, wpc, window, 0)

  return k


def _zeros_buf(zb):
  return jnp.zeros((zb, H), jnp.float32)


def _sc_gather(table, idx, m_rows, chunk, kq):
  return _make_gather(m_rows, chunk, kq)(table, idx)


def _segsum_edges(t, idx_ji):
  return _make_scatter(T, 8000, 10, 125, 80, 5)(t, idx_ji, _zeros_buf(125))


def _segsum_nodes(t, i):
  out = _make_scatter(E, 5120, 1, 160, 80, 5)(t, i, _zeros_buf(160))
  return out[:N]


def kernel(x, rbf, sbf, i, j, idx_kj, idx_ji, params):
  i = i.astype(jnp.int32)
  j = j.astype(jnp.int32)
  idx_kj = idx_kj.astype(jnp.int32)
  idx_ji = idx_ji.astype(jnp.int32)

  a12 = _node_embed(x, params["emb"])
  ij = jnp.concatenate([i, j + N])
  s12 = _sc_gather(a12, ij, 2 * E, 40, 10)
  m, t_out = _edge_embed(s12, rbf, params["emb"],
                         params["out"][0]["rbf"]["w"])
  total = jnp.zeros((1, 1), jnp.float32)
  for b in range(NBLK):
    pb = params["inter"][b]
    xji, xkjr = _inter1(m, rbf, pb)
    xg = _sc_gather(xkjr, idx_kj, T, 80, 5)
    t = _inter2(xg, sbf, pb)
    agg = _segsum_edges(t, idx_ji)
    total = total + _out_node(_segsum_nodes(t_out, i), params["out"][b])
    m, t_out = _inter3(xji, agg, m, rbf, pb,
                       params["out"][b + 1]["rbf"]["w"])
  total = total + _out_node(_segsum_nodes(t_out, i), params["out"][NBLK])
  return total.reshape((1,))


# speedup vs baseline: 2.2085x; 1.0031x over previous
import functools

import jax
import jax.numpy as jnp
from jax import lax
from jax.experimental import pallas as pl
from jax.experimental.pallas import tpu as pltpu
from jax.experimental.pallas import tpu_sc as plsc

N = 10000
E = 160000
T = 64000
H = 128
NR = 6
NB = 8
NBLK = 2
SBF_DIM = 42

NC = 2
NSUB = 16
NW = NC * NSUB

BE = 1280
BT = 1000

_silu = jax.nn.silu


def _dot(a, b):
  return jnp.dot(a, b, preferred_element_type=jnp.float32)


def _node_embed_body(x_ref, f1w, f1b, bng, bnb, f2w, f2b, w1, w2, a_ref):
  x = x_ref[...]
  h1 = _dot(x, f1w[...]) + f1b[...]
  mu = jnp.mean(h1, axis=0, keepdims=True)
  var = jnp.mean((h1 - mu) ** 2, axis=0, keepdims=True)
  h1 = (h1 - mu) / jnp.sqrt(var + 1e-5) * bng[...] + bnb[...]
  h1 = jnp.maximum(h1, 0.0)
  h = jnp.maximum(_dot(h1, f2w[...]) + f2b[...], 0.0)
  a_ref[:N, :] = _dot(h, w1[...])
  a_ref[N:, :] = _dot(h, w2[...])


def _node_embed(x, p):
  return pl.pallas_call(
      _node_embed_body,
      out_shape=jax.ShapeDtypeStruct((2 * N, H), jnp.float32),
  )(x, p["f1"]["w"], p["f1"]["b"].reshape(1, -1),
    p["bng"].reshape(1, -1), p["bnb"].reshape(1, -1),
    p["f2"]["w"], p["f2"]["b"].reshape(1, -1),
    p["lin"]["w"][:H], p["lin"]["w"][H:2 * H])


def _edge_embed_body(s1_ref, s2_ref, rbf_ref, wr, br, w3, bl, wro,
                     m_ref, t0_ref):
  rbf = rbf_ref[...]
  r = _silu(_dot(rbf, wr[...]) + br[...])
  m = _silu(s1_ref[...] + s2_ref[...] + _dot(r, w3[...]) + bl[...])
  m_ref[...] = m
  t0_ref[...] = _dot(rbf, wro[...]) * m


def _edge_embed(s12, rbf, p, wro):
  eb = lambda i: (i, 0)
  full = lambda i: (0, 0)
  return pl.pallas_call(
      _edge_embed_body,
      grid=(E // BE,),
      in_specs=[
          pl.BlockSpec((BE, H), eb),
          pl.BlockSpec((BE, H), lambda i: (i + E // BE, 0)),
          pl.BlockSpec((BE, NR), eb),
          pl.BlockSpec((NR, H), full),
          pl.BlockSpec((1, H), full),
          pl.BlockSpec((H, H), full),
          pl.BlockSpec((1, H), full),
          pl.BlockSpec((NR, H), full),
      ],
      out_specs=(pl.BlockSpec((BE, H), eb), pl.BlockSpec((BE, H), eb)),
      out_shape=(jax.ShapeDtypeStruct((E, H), jnp.float32),
                 jax.ShapeDtypeStruct((E, H), jnp.float32)),
  )(s12, s12, rbf, p["rbf"]["w"], p["rbf"]["b"].reshape(1, -1),
    p["lin"]["w"][2 * H:], p["lin"]["b"].reshape(1, -1), wro)


def _out_node_body(np_ref, l1w, l1b, l2w, l2b, l3w, l3b, wout, out_ref):
  node = np_ref[...]
  node = _silu(_dot(node, l1w[...]) + l1b[...])
  node = _silu(_dot(node, l2w[...]) + l2b[...])
  node = _silu(_dot(node, l3w[...]) + l3b[...])
  tot = jnp.sum(node, axis=0, keepdims=True)
  out_ref[...] = _dot(tot, wout[...])


def _out_node(node, p):
  ls = p["lins"]
  return pl.pallas_call(
      _out_node_body,
      out_shape=jax.ShapeDtypeStruct((1, 1), jnp.float32),
  )(node,
    ls[0]["w"], ls[0]["b"].reshape(1, -1),
    ls[1]["w"], ls[1]["b"].reshape(1, -1),
    ls[2]["w"], ls[2]["b"].reshape(1, -1),
    p["out"])


def _inter1_body(m_ref, rbf_ref, wkj, bkj, wji, bji, wr, xji_ref, xkjr_ref):
  m = m_ref[...]
  xji_ref[...] = _silu(_dot(m, wji[...]) + bji[...])
  rbfp = _dot(rbf_ref[...], wr[...])
  xkjr_ref[...] = _silu(_dot(m, wkj[...]) + bkj[...]) * rbfp


def _inter1(m, rbf, p):
  eb = lambda i: (i, 0)
  full = lambda i: (0, 0)
  return pl.pallas_call(
      _inter1_body,
      grid=(E // BE,),
      in_specs=[
          pl.BlockSpec((BE, H), eb),
          pl.BlockSpec((BE, NR), eb),
          pl.BlockSpec((H, H), full),
          pl.BlockSpec((1, H), full),
          pl.BlockSpec((H, H), full),
          pl.BlockSpec((1, H), full),
          pl.BlockSpec((NR, H), full),
      ],
      out_specs=(pl.BlockSpec((BE, H), eb), pl.BlockSpec((BE, H), eb)),
      out_shape=(jax.ShapeDtypeStruct((E, H), jnp.float32),
                 jax.ShapeDtypeStruct((E, H), jnp.float32)),
  )(m, rbf, p["kj"]["w"], p["kj"]["b"].reshape(1, -1),
    p["ji"]["w"], p["ji"]["b"].reshape(1, -1), p["rbf"]["w"])


def _inter2_body(xg_ref, sbf_ref, ws, wt, t_ref):
  sp = _dot(sbf_ref[...], ws[...])
  xg = xg_ref[...]
  acc = sp[:, 0:1] * _dot(xg, wt[0])
  for b in range(1, NB):
    acc = acc + sp[:, b:b + 1] * _dot(xg, wt[b])
  t_ref[...] = acc


def _inter2(xg, sbf, p):
  wt = jnp.transpose(p["W"], (1, 2, 0))
  tb = lambda i: (i, 0)
  return pl.pallas_call(
      _inter2_body,
      grid=(T // BT,),
      in_specs=[
          pl.BlockSpec((BT, H), tb),
          pl.BlockSpec((BT, SBF_DIM), tb),
          pl.BlockSpec((SBF_DIM, NB), lambda i: (0, 0)),
          pl.BlockSpec((NB, H, H), lambda i: (0, 0, 0)),
      ],
      out_specs=pl.BlockSpec((BT, H), tb),
      out_shape=jax.ShapeDtypeStruct((T, H), jnp.float32),
  )(xg, sbf, p["sbf"]["w"], wt)


def _inter3_body(xji_ref, agg_ref, m_ref, rbf_ref,
                 b1w, b1b, b2w, b2b, lw, lb,
                 a1w, a1b, a2w, a2b, a3w, a3b, a4w, a4b, wro,
                 mo_ref, tn_ref):
  h = xji_ref[...] + agg_ref[...]
  h = h + _silu(_dot(_silu(_dot(h, b1w[...]) + b1b[...]), b2w[...]) + b2b[...])
  h = _silu(_dot(h, lw[...]) + lb[...]) + m_ref[...]
  h = h + _silu(_dot(_silu(_dot(h, a1w[...]) + a1b[...]), a2w[...]) + a2b[...])
  h = h + _silu(_dot(_silu(_dot(h, a3w[...]) + a3b[...]), a4w[...]) + a4b[...])
  mo_ref[...] = h
  tn_ref[...] = _dot(rbf_ref[...], wro[...]) * h


def _inter3(xji, agg, m, rbf, p, wro):
  eb = lambda i: (i, 0)
  full = lambda i: (0, 0)
  wspec = pl.BlockSpec((H, H), full)
  bspec = pl.BlockSpec((1, H), full)
  bef = p["before"][0]
  af0, af1 = p["after"][0], p["after"][1]
  return pl.pallas_call(
      _inter3_body,
      grid=(E // BE,),
      in_specs=[
          pl.BlockSpec((BE, H), eb),
          pl.BlockSpec((BE, H), eb),
          pl.BlockSpec((BE, H), eb),
          pl.BlockSpec((BE, NR), eb),
          wspec, bspec, wspec, bspec, wspec, bspec,
          wspec, bspec, wspec, bspec, wspec, bspec, wspec, bspec,
          pl.BlockSpec((NR, H), full),
      ],
      out_specs=(pl.BlockSpec((BE, H), eb), pl.BlockSpec((BE, H), eb)),
      out_shape=(jax.ShapeDtypeStruct((E, H), jnp.float32),
                 jax.ShapeDtypeStruct((E, H), jnp.float32)),
  )(xji, agg, m, rbf,
    bef["l1"]["w"], bef["l1"]["b"].reshape(1, -1),
    bef["l2"]["w"], bef["l2"]["b"].reshape(1, -1),
    p["lin"]["w"], p["lin"]["b"].reshape(1, -1),
    af0["l1"]["w"], af0["l1"]["b"].reshape(1, -1),
    af0["l2"]["w"], af0["l2"]["b"].reshape(1, -1),
    af1["l1"]["w"], af1["l1"]["b"].reshape(1, -1),
    af1["l2"]["w"], af1["l2"]["b"].reshape(1, -1),
    wro)


@functools.lru_cache(maxsize=None)
def _sc_mesh():
  return plsc.VectorSubcoreMesh(core_axis_name="c", subcore_axis_name="s")


@functools.lru_cache(maxsize=None)
def _make_gather(m_rows, chunk, kq):
  per_tile = m_rows // NW
  n_chunks = per_tile // chunk
  assert n_chunks % kq == 0

  @functools.partial(
      pl.kernel,
      out_type=jax.ShapeDtypeStruct((m_rows, H), jnp.float32),
      mesh=_sc_mesh(),
      scratch_types=[
          pltpu.VMEM((per_tile,), jnp.int32),
          pltpu.VMEM((2, kq, chunk, H), jnp.float32),
          pltpu.SemaphoreType.DMA,
          pltpu.SemaphoreType.DMA,
      ],
  )
  def k(table_hbm, idx_hbm, out_hbm, idx_all, rows, sem_g, sem_w):
    wid = lax.axis_index("s") * NC + lax.axis_index("c")
    base = wid * per_tile
    pltpu.sync_copy(idx_hbm.at[pl.ds(base, per_tile)], idx_all)
    n_groups = n_chunks // kq

    def group(g, carry):
      s = g % 2

      @pl.when(g >= 2)
      def _():
        for _b in range(kq):
          pltpu.make_async_copy(
              rows.at[0, 0], out_hbm.at[pl.ds(base, chunk)], sem_w).wait()

      gd = []
      for b in range(kq):
        off = (g * kq + b) * chunk
        gd.append(pltpu.async_copy(
            table_hbm.at[idx_all.at[pl.ds(off, chunk)]], rows.at[s, b],
            sem_g))
      for d in gd:
        d.wait()
      for b in range(kq):
        off = (g * kq + b) * chunk
        pltpu.async_copy(
            rows.at[s, b], out_hbm.at[pl.ds(base + off, chunk)], sem_w)
      return carry

    lax.fori_loop(0, n_groups, group, 0)
    for _b in range(min(2, n_groups) * kq):
      pltpu.make_async_copy(
          rows.at[0, 0], out_hbm.at[pl.ds(base, chunk)], sem_w).wait()

  return k


@functools.lru_cache(maxsize=None)
def _make_scatter(m_rows, win, wpc, zb, chunk, kb):
  per_tile = m_rows // NSUB
  n_groups = per_tile // 16
  cap = per_tile + chunk + 16
  wt_count = NSUB if (win // NSUB) % 8 == 0 else NSUB // 2
  rpt = win // wt_count
  n_zero = rpt // zb
  assert rpt % zb == 0 and per_tile % 16 == 0 and chunk % 16 == 0

  @functools.partial(
      pl.kernel,
      out_type=jax.ShapeDtypeStruct((2 * wpc * win, H), jnp.float32),
      mesh=_sc_mesh(),
      compiler_params=pltpu.CompilerParams(needs_layout_passes=False),
      scratch_types=[
          pltpu.VMEM_SHARED((win + 8, H), jnp.float32),
          pltpu.VMEM((per_tile,), jnp.int32),
          pltpu.VMEM((cap,), jnp.int32),
          pltpu.VMEM((cap,), jnp.int32),
          pltpu.VMEM((chunk,), jnp.int32),
          pltpu.VMEM((kb, chunk, H), jnp.float32),
          pltpu.SemaphoreType.DMA,
          pltpu.SemaphoreType.DMA,
      ],
  )
  def k(src_hbm, idx_hbm, zeros_hbm, out_hbm, shared, idx_all,
        dest_list, tid_list, dest_v, bufs, sem_g, sem_z):
    cid = lax.axis_index("c")
    sid = lax.axis_index("s")
    src_base = sid * per_tile
    pltpu.sync_copy(idx_hbm.at[pl.ds(src_base, per_tile)], idx_all)
    iota16 = lax.iota(jnp.int32, 16)

    def window(w, carry):
      lo = (cid * wpc + w) * win
      out_base = (cid * wpc + w) * win

      @pl.when(sid < wt_count)
      def _():
        for z in range(n_zero):
          pltpu.async_copy(
              zeros_hbm, shared.at[pl.ds(sid * rpt + z * zb, zb)], sem_z)

      def grp(g, cnt):
        v = idx_all[pl.ds(g * 16, 16)] - lo
        okb = (v >= 0) & (v < win)
        ok = okb.astype(jnp.int32)
        pre = plsc.cumsum(ok)
        pos = jnp.where(okb, cnt + pre - ok, (cap - 16) + iota16)
        plsc.store_scatter(dest_list, [pos], v)
        plsc.store_scatter(tid_list, [pos], iota16 + (src_base + g * 16))
        return cnt + plsc.all_reduce_population_count(okb)[0]

      cnt = lax.fori_loop(0, n_groups, grp, 0)

      @pl.when(sid < wt_count)
      def _():
        for z in range(n_zero):
          pltpu.make_async_copy(
              zeros_hbm, shared.at[pl.ds(sid * rpt + z * zb, zb)],
              sem_z).wait()

      plsc.subcore_barrier()
      for kp in range(chunk // 16):
        pos = cnt + kp * 16 + iota16
        plsc.store_scatter(dest_list, [pos], jnp.full((16,), win, jnp.int32))
        plsc.store_scatter(tid_list, [pos],
                           jnp.full((16,), src_base, jnp.int32))

      n_slots = (per_tile + chunk - 1) // chunk
      n_cgroups = (n_slots + kb - 1) // kb

      def cgroup(g2, cc):
        for b in range(kb):
          c = g2 * kb + b
          if True:
            @pl.when(c * chunk < cnt)
            def _(c=c, b=b):
              off = c * chunk
              pltpu.async_copy(
                  src_hbm.at[tid_list.at[pl.ds(off, chunk)]],
                  bufs.at[b], sem_g)

        for b in range(kb):
          c = g2 * kb + b
          if True:
            @pl.when(c * chunk < cnt)
            def _(c=c, b=b):
              pltpu.make_async_copy(
                  src_hbm.at[pl.ds(0, chunk)], bufs.at[b], sem_g).wait()

        for b in range(kb):
          c = g2 * kb + b
          if True:
            @pl.when(c * chunk < cnt)
            def _(c=c, b=b):
              off = c * chunk
              for q in range(chunk // 16):
                dest_v[pl.ds(q * 16, 16)] = dest_list[pl.ds(off + q * 16, 16)]
              pltpu.sync_copy(bufs.at[b], shared.at[dest_v], add=True)

        return cc

      lax.fori_loop(0, n_cgroups, cgroup, 0)
      plsc.subcore_barrier()

      @pl.when(sid < wt_count)
      def _():
        pltpu.sync_copy(shared.at[pl.ds(sid * rpt, rpt)],
                        out_hbm.at[pl.ds(out_base + sid * rpt, rpt)])

      plsc.subcore_barrier()
      return carry

    lax.fori_loop(0, wpc, window, 0)

  return k


def _zeros_buf(zb):
  return jnp.zeros((zb, H), jnp.float32)


def _sc_gather(table, idx, m_rows, chunk, kq):
  return _make_gather(m_rows, chunk, kq)(table, idx)


def _segsum_edges(t, idx_ji):
  return _make_scatter(T, 8000, 10, 125, 80, 5)(t, idx_ji, _zeros_buf(125))


def _segsum_nodes(t, i):
  out = _make_scatter(E, 5120, 1, 160, 80, 5)(t, i, _zeros_buf(160))
  return out[:N]


def kernel(x, rbf, sbf, i, j, idx_kj, idx_ji, params):
  i = i.astype(jnp.int32)
  j = j.astype(jnp.int32)
  idx_kj = idx_kj.astype(jnp.int32)
  idx_ji = idx_ji.astype(jnp.int32)

  a12 = _node_embed(x, params["emb"])
  ij = jnp.concatenate([i, j + N])
  s12 = _sc_gather(a12, ij, 2 * E, 40, 10)
  m, t_out = _edge_embed(s12, rbf, params["emb"],
                         params["out"][0]["rbf"]["w"])
  total = jnp.zeros((1, 1), jnp.float32)
  for b in range(NBLK):
    pb = params["inter"][b]
    xji, xkjr = _inter1(m, rbf, pb)
    xg = _sc_gather(xkjr, idx_kj, T, 80, 5)
    t = _inter2(xg, sbf, pb)
    agg = _segsum_edges(t, idx_ji)
    total = total + _out_node(_segsum_nodes(t_out, i), params["out"][b])
    m, t_out = _inter3(xji, agg, m, rbf, pb,
                       params["out"][b + 1]["rbf"]["w"])
  total = total + _out_node(_segsum_nodes(t_out, i), params["out"][NBLK])
  return total.reshape((1,))
